# Initial kernel scaffold; baseline (speedup 1.0000x reference)
import jax
import jax.numpy as jnp
from jax.experimental import pallas as pl


def kernel(x, edge_index, W1, b1, W2, b2):
    raise NotImplementedError("write your pallas kernel here")


# speedup vs baseline: 27.5730x; 27.5730x over previous
import functools

import jax
import jax.numpy as jnp
from jax import lax
from jax.experimental import pallas as pl
from jax.experimental.pallas import tpu as pltpu
from jax.experimental.pallas import tpu_sc as plsc

N_NODES = 10000
N_EDGES = 320000
IN_DIM = 128
HID_DIM = 256
OUT_DIM = 64

NC = 2
NS = 16
NW = NC * NS
K = 128
EPW = N_EDGES // NW
CH = (EPW + K - 1) // K
EPW_PAD = CH * K
NR = 10112
STRIPE = NR // NS
DEG_W = 16


def _scatter_add_kernel(feat_w):
    mesh = plsc.VectorSubcoreMesh(core_axis_name="c", subcore_axis_name="s")

    @functools.partial(
        pl.kernel,
        out_type=jax.ShapeDtypeStruct((NC, N_NODES, feat_w), jnp.float32),
        mesh=mesh,
        compiler_params=pltpu.CompilerParams(
            use_tc_tiling_on_sc=(feat_w % 128 == 0)),
        scratch_types=[
            pltpu.VMEM((CH, K), jnp.int32),
            pltpu.VMEM((CH, K), jnp.int32),
            pltpu.VMEM((K, feat_w), jnp.float32),
            pltpu.VMEM_SHARED((NR, feat_w), jnp.float32),
            pltpu.SemaphoreType.DMA,
        ],
    )
    def k(table_hbm, srcp_hbm, dstp_hbm, out_hbm, src_v, dst_v, rows_v, acc, sem):
        cid = lax.axis_index("c")
        sid = lax.axis_index("s")
        wid = cid * NS + sid

        pltpu.sync_copy(srcp_hbm.at[wid], src_v)
        pltpu.sync_copy(dstp_hbm.at[wid], dst_v)

        z = jnp.zeros((16,), jnp.float32)

        def zrow(i, _):
            for c in range(feat_w // 16):
                rows_v[i, pl.ds(c * 16, 16)] = z
            return 0

        lax.fori_loop(0, K, zrow, 0)
        base = sid * STRIPE
        n_full = STRIPE // K
        for t in range(n_full):
            pltpu.sync_copy(rows_v, acc.at[pl.ds(base + t * K, K)])
        rem = STRIPE - n_full * K
        if rem:
            pltpu.sync_copy(rows_v.at[pl.ds(0, rem)],
                            acc.at[pl.ds(base + n_full * K, rem)])
        plsc.subcore_barrier()

        def body(j, _):
            pltpu.async_copy(table_hbm.at[src_v.at[j]], rows_v, sem).wait()
            pltpu.sync_copy(rows_v, acc.at[dst_v.at[j]], add=True)
            return 0

        lax.fori_loop(0, CH, body, 0)
        plsc.subcore_barrier()

        @pl.when(sid < NS - 1)
        def _():
            pltpu.sync_copy(acc.at[pl.ds(base, STRIPE)],
                            out_hbm.at[cid, pl.ds(base, STRIPE)])

        @pl.when(sid == NS - 1)
        def _():
            last = N_NODES - (NS - 1) * STRIPE
            pltpu.sync_copy(acc.at[pl.ds((NS - 1) * STRIPE, last)],
                            out_hbm.at[cid, pl.ds((NS - 1) * STRIPE, last)])

    return k


def _deg_kernel():
    mesh = plsc.VectorSubcoreMesh(core_axis_name="c", subcore_axis_name="s")

    @functools.partial(
        pl.kernel,
        out_type=jax.ShapeDtypeStruct((NC, N_NODES, DEG_W), jnp.float32),
        mesh=mesh,
        scratch_types=[
            pltpu.VMEM((CH, K), jnp.int32),
            pltpu.VMEM((K, DEG_W), jnp.float32),
            pltpu.VMEM_SHARED((NR, DEG_W), jnp.float32),
        ],
    )
    def k(dstp_hbm, out_hbm, dst_v, ones_v, acc):
        cid = lax.axis_index("c")
        sid = lax.axis_index("s")
        wid = cid * NS + sid

        pltpu.sync_copy(dstp_hbm.at[wid], dst_v)

        z = jnp.zeros((16,), jnp.float32)
        one = jnp.ones((16,), jnp.float32)

        def fill(val):
            def row(i, _):
                ones_v[i, pl.ds(0, 16)] = val
                return 0
            return row

        lax.fori_loop(0, K, fill(z), 0)
        base = sid * STRIPE
        n_full = STRIPE // K
        for t in range(n_full):
            pltpu.sync_copy(ones_v, acc.at[pl.ds(base + t * K, K)])
        rem = STRIPE - n_full * K
        if rem:
            pltpu.sync_copy(ones_v.at[pl.ds(0, rem)],
                            acc.at[pl.ds(base + n_full * K, rem)])
        lax.fori_loop(0, K, fill(one), 0)
        plsc.subcore_barrier()

        def body(j, _):
            pltpu.sync_copy(ones_v, acc.at[dst_v.at[j]], add=True)
            return 0

        lax.fori_loop(0, CH, body, 0)
        plsc.subcore_barrier()

        @pl.when(sid < NS - 1)
        def _():
            pltpu.sync_copy(acc.at[pl.ds(base, STRIPE)],
                            out_hbm.at[cid, pl.ds(base, STRIPE)])

        @pl.when(sid == NS - 1)
        def _():
            last = N_NODES - (NS - 1) * STRIPE
            pltpu.sync_copy(acc.at[pl.ds((NS - 1) * STRIPE, last)],
                            out_hbm.at[cid, pl.ds((NS - 1) * STRIPE, last)])

    return k


_R = 1000
_G = N_NODES // _R


def _tc_scale_kernel(deg, x):
    def body(d0, d1, xr, o):
        degt = d0[0, :, 0:1] + d1[0, :, 0:1] + 1.0
        o[...] = xr[...] * lax.rsqrt(degt)

    return pl.pallas_call(
        body,
        grid=(_G,),
        in_specs=[
            pl.BlockSpec((1, _R, DEG_W), lambda i: (0, i, 0)),
            pl.BlockSpec((1, _R, DEG_W), lambda i: (1, i, 0)),
            pl.BlockSpec((_R, IN_DIM), lambda i: (i, 0)),
        ],
        out_specs=pl.BlockSpec((_R, IN_DIM), lambda i: (i, 0)),
        out_shape=jax.ShapeDtypeStruct((N_NODES, IN_DIM), jnp.float32),
    )(deg, deg, x)


def _tc_mid_kernel(p, xs, deg, W1, b1, W2):
    def body(p0, p1, xsr, d0, d1, w1, b1r, w2, o):
        degt = d0[0, :, 0:1] + d1[0, :, 0:1] + 1.0
        dis = lax.rsqrt(degt)
        pr = (p0[0] + p1[0] + xsr[...]) * dis
        h = jnp.maximum(
            jnp.dot(pr, w1[...], preferred_element_type=jnp.float32) + b1r[...],
            0.0)
        m = jnp.dot(h, w2[...], preferred_element_type=jnp.float32)
        o[...] = m * dis

    return pl.pallas_call(
        body,
        grid=(_G,),
        in_specs=[
            pl.BlockSpec((1, _R, IN_DIM), lambda i: (0, i, 0)),
            pl.BlockSpec((1, _R, IN_DIM), lambda i: (1, i, 0)),
            pl.BlockSpec((_R, IN_DIM), lambda i: (i, 0)),
            pl.BlockSpec((1, _R, DEG_W), lambda i: (0, i, 0)),
            pl.BlockSpec((1, _R, DEG_W), lambda i: (1, i, 0)),
            pl.BlockSpec((IN_DIM, HID_DIM), lambda i: (0, 0)),
            pl.BlockSpec((1, HID_DIM), lambda i: (0, 0)),
            pl.BlockSpec((HID_DIM, OUT_DIM), lambda i: (0, 0)),
        ],
        out_specs=pl.BlockSpec((_R, OUT_DIM), lambda i: (i, 0)),
        out_shape=jax.ShapeDtypeStruct((N_NODES, OUT_DIM), jnp.float32),
    )(p, p, xs, deg, deg, W1, b1.reshape(1, HID_DIM), W2)


def _tc_final_kernel(q, ms, deg, b2):
    def body(q0, q1, msr, d0, d1, b2r, o):
        degt = d0[0, :, 0:1] + d1[0, :, 0:1] + 1.0
        dis = lax.rsqrt(degt)
        t = (q0[0] + q1[0] + msr[...]) * dis + b2r[...]
        mx = jnp.max(t, axis=1, keepdims=True)
        e = jnp.exp(t - mx)
        lse = jnp.log(jnp.sum(e, axis=1, keepdims=True))
        o[...] = t - mx - lse

    return pl.pallas_call(
        body,
        grid=(_G,),
        in_specs=[
            pl.BlockSpec((1, _R, OUT_DIM), lambda i: (0, i, 0)),
            pl.BlockSpec((1, _R, OUT_DIM), lambda i: (1, i, 0)),
            pl.BlockSpec((_R, OUT_DIM), lambda i: (i, 0)),
            pl.BlockSpec((1, _R, DEG_W), lambda i: (0, i, 0)),
            pl.BlockSpec((1, _R, DEG_W), lambda i: (1, i, 0)),
            pl.BlockSpec((1, OUT_DIM), lambda i: (0, 0)),
        ],
        out_specs=pl.BlockSpec((_R, OUT_DIM), lambda i: (i, 0)),
        out_shape=jax.ShapeDtypeStruct((N_NODES, OUT_DIM), jnp.float32),
    )(q, q, ms, deg, deg, b2.reshape(1, OUT_DIM))


def kernel(x, edge_index, W1, b1, W2, b2):
    src = edge_index[0].astype(jnp.int32)
    dst = edge_index[1].astype(jnp.int32)

    n_pad = EPW_PAD - EPW
    pad_src = (jnp.arange(n_pad, dtype=jnp.int32) * 89) % N_NODES
    pad_dst = N_NODES + (jnp.arange(n_pad, dtype=jnp.int32) % (NR - N_NODES))
    srcp = jnp.concatenate(
        [src.reshape(NW, EPW), jnp.broadcast_to(pad_src, (NW, n_pad))], axis=1
    ).reshape(NW, CH, K)
    dstp = jnp.concatenate(
        [dst.reshape(NW, EPW), jnp.broadcast_to(pad_dst, (NW, n_pad))], axis=1
    ).reshape(NW, CH, K)

    deg = _deg_kernel()(dstp)
    xs = _tc_scale_kernel(deg, x)
    p = _scatter_add_kernel(IN_DIM)(xs, srcp, dstp)
    ms = _tc_mid_kernel(p, xs, deg, W1, b1, W2)
    q = _scatter_add_kernel(OUT_DIM)(ms, srcp, dstp)
    return _tc_final_kernel(q, ms, deg, b2)


# speedup vs baseline: 32.0390x; 1.1620x over previous
import functools

import jax
import jax.numpy as jnp
from jax import lax
from jax.experimental import pallas as pl
from jax.experimental.pallas import tpu as pltpu
from jax.experimental.pallas import tpu_sc as plsc

N_NODES = 10000
N_EDGES = 320000
IN_DIM = 128
HID_DIM = 256
OUT_DIM = 64

NC = 2
NS = 16
NW = NC * NS
K = 128
EPW = N_EDGES // NW
CH = 80
CH_IDX = CH + 2
EPW_PAD = CH_IDX * K
NR = 10112
STRIPE = NR // NS
DEG_W = 4


FW = 64


def _scatter_add_kernel(nph):
    mesh = plsc.VectorSubcoreMesh(core_axis_name="c", subcore_axis_name="s")

    @functools.partial(
        pl.kernel,
        out_type=jax.ShapeDtypeStruct((NC, nph, N_NODES, FW), jnp.float32),
        mesh=mesh,
        compiler_params=pltpu.CompilerParams(use_tc_tiling_on_sc=False),
        scratch_types=[
            pltpu.VMEM((CH_IDX, K), jnp.int32),
            pltpu.VMEM((CH_IDX, K), jnp.int32),
            pltpu.VMEM((2, K, FW), jnp.float32),
            pltpu.VMEM_SHARED((NR, FW), jnp.float32),
            pltpu.SemaphoreType.DMA,
            pltpu.SemaphoreType.DMA,
        ],
    )
    def k(table_hbm, srcp_hbm, dstp_hbm, out_hbm, src_v, dst_v, rows_v, acc,
          sem0, sem1):
        cid = lax.axis_index("c")
        sid = lax.axis_index("s")
        wid = cid * NS + sid

        pltpu.sync_copy(srcp_hbm.at[wid], src_v)
        pltpu.sync_copy(dstp_hbm.at[wid], dst_v)

        z = jnp.zeros((16,), jnp.float32)

        def zrow(i, _):
            for c in range(FW // 16):
                rows_v[0, i, pl.ds(c * 16, 16)] = z
            return 0

        lax.fori_loop(0, K, zrow, 0)
        base = sid * STRIPE
        sems = (sem0, sem1)
        last = N_NODES - (NS - 1) * STRIPE

        for ph in range(nph):
            if ph:
                lax.fori_loop(0, K, zrow, 0)
            n_full = STRIPE // K
            for t in range(n_full):
                pltpu.sync_copy(rows_v.at[0], acc.at[pl.ds(base + t * K, K)])
            rem = STRIPE - n_full * K
            if rem:
                pltpu.sync_copy(rows_v.at[0, pl.ds(0, rem)],
                                acc.at[pl.ds(base + n_full * K, rem)])
            plsc.subcore_barrier()

            tab = table_hbm.at[ph]
            pltpu.async_copy(tab.at[src_v.at[0]], rows_v.at[0], sem0)
            pltpu.async_copy(tab.at[src_v.at[1]], rows_v.at[1], sem1)

            def body(i, _):
                for b in range(2):
                    j = 2 * i + b
                    pltpu.make_async_copy(
                        tab.at[src_v.at[j]], rows_v.at[b], sems[b]).wait()
                    pltpu.sync_copy(rows_v.at[b], acc.at[dst_v.at[j]],
                                    add=True)
                    pltpu.async_copy(
                        tab.at[src_v.at[j + 2]], rows_v.at[b], sems[b])
                return 0

            lax.fori_loop(0, CH // 2, body, 0)
            for b in range(2):
                pltpu.make_async_copy(
                    tab.at[src_v.at[CH + b]], rows_v.at[b], sems[b]).wait()
            plsc.subcore_barrier()

            @pl.when(sid < NS - 1)
            def _():
                pltpu.sync_copy(acc.at[pl.ds(base, STRIPE)],
                                out_hbm.at[cid, ph, pl.ds(base, STRIPE)])

            @pl.when(sid == NS - 1)
            def _():
                pltpu.sync_copy(
                    acc.at[pl.ds((NS - 1) * STRIPE, last)],
                    out_hbm.at[cid, ph, pl.ds((NS - 1) * STRIPE, last)])

            if ph + 1 < nph:
                plsc.subcore_barrier()

    return k


def _deg_kernel():
    mesh = plsc.VectorSubcoreMesh(core_axis_name="c", subcore_axis_name="s")

    @functools.partial(
        pl.kernel,
        out_type=jax.ShapeDtypeStruct((NC, N_NODES, DEG_W), jnp.float32),
        mesh=mesh,
        scratch_types=[
            pltpu.VMEM((CH_IDX, K), jnp.int32),
            pltpu.VMEM((2, K, DEG_W), jnp.float32),
            pltpu.VMEM_SHARED((NR, DEG_W), jnp.float32),
        ],
    )
    def k(zo_hbm, dstp_hbm, out_hbm, dst_v, zo_v, acc):
        cid = lax.axis_index("c")
        sid = lax.axis_index("s")
        wid = cid * NS + sid

        pltpu.sync_copy(dstp_hbm.at[wid], dst_v)
        pltpu.sync_copy(zo_hbm, zo_v)
        ones_v = zo_v.at[1]

        base = sid * STRIPE
        n_full = STRIPE // K
        for t in range(n_full):
            pltpu.sync_copy(zo_v.at[0], acc.at[pl.ds(base + t * K, K)])
        rem = STRIPE - n_full * K
        if rem:
            pltpu.sync_copy(zo_v.at[0, pl.ds(0, rem)],
                            acc.at[pl.ds(base + n_full * K, rem)])
        plsc.subcore_barrier()

        def body(j, _):
            pltpu.sync_copy(ones_v, acc.at[dst_v.at[j]], add=True)
            return 0

        lax.fori_loop(0, CH, body, 0)
        plsc.subcore_barrier()

        @pl.when(sid < NS - 1)
        def _():
            pltpu.sync_copy(acc.at[pl.ds(base, STRIPE)],
                            out_hbm.at[cid, pl.ds(base, STRIPE)])

        @pl.when(sid == NS - 1)
        def _():
            last = N_NODES - (NS - 1) * STRIPE
            pltpu.sync_copy(acc.at[pl.ds((NS - 1) * STRIPE, last)],
                            out_hbm.at[cid, pl.ds((NS - 1) * STRIPE, last)])

    return k


_R = 1000
_G = N_NODES // _R


def _tc_scale_kernel(deg, x):
    def body(d0, d1, xr, o):
        degt = d0[0, :, 0:1] + d1[0, :, 0:1] + 1.0
        dis = lax.rsqrt(degt)
        o[0] = xr[:, :FW] * dis
        o[1] = xr[:, FW:] * dis

    return pl.pallas_call(
        body,
        grid=(_G,),
        in_specs=[
            pl.BlockSpec((1, _R, DEG_W), lambda i: (0, i, 0)),
            pl.BlockSpec((1, _R, DEG_W), lambda i: (1, i, 0)),
            pl.BlockSpec((_R, IN_DIM), lambda i: (i, 0)),
        ],
        out_specs=pl.BlockSpec((2, _R, FW), lambda i: (0, i, 0)),
        out_shape=jax.ShapeDtypeStruct((2, N_NODES, FW), jnp.float32),
    )(deg, deg, x)


def _tc_mid_kernel(p, xs, deg, W1, b1, W2):
    def body(p0, p1, xsr, d0, d1, w1, b1r, w2, o):
        degt = d0[0, :, 0:1] + d1[0, :, 0:1] + 1.0
        dis = lax.rsqrt(degt)
        pr = jnp.concatenate(
            [p0[0, 0] + p1[0, 0] + xsr[0], p0[0, 1] + p1[0, 1] + xsr[1]],
            axis=1) * dis
        h = jnp.maximum(
            jnp.dot(pr, w1[...], preferred_element_type=jnp.float32) + b1r[...],
            0.0)
        m = jnp.dot(h, w2[...], preferred_element_type=jnp.float32)
        o[0] = m * dis

    return pl.pallas_call(
        body,
        grid=(_G,),
        in_specs=[
            pl.BlockSpec((1, 2, _R, FW), lambda i: (0, 0, i, 0)),
            pl.BlockSpec((1, 2, _R, FW), lambda i: (1, 0, i, 0)),
            pl.BlockSpec((2, _R, FW), lambda i: (0, i, 0)),
            pl.BlockSpec((1, _R, DEG_W), lambda i: (0, i, 0)),
            pl.BlockSpec((1, _R, DEG_W), lambda i: (1, i, 0)),
            pl.BlockSpec((IN_DIM, HID_DIM), lambda i: (0, 0)),
            pl.BlockSpec((1, HID_DIM), lambda i: (0, 0)),
            pl.BlockSpec((HID_DIM, OUT_DIM), lambda i: (0, 0)),
        ],
        out_specs=pl.BlockSpec((1, _R, OUT_DIM), lambda i: (0, i, 0)),
        out_shape=jax.ShapeDtypeStruct((1, N_NODES, OUT_DIM), jnp.float32),
    )(p, p, xs, deg, deg, W1, b1.reshape(1, HID_DIM), W2)


def _tc_final_kernel(q, ms, deg, b2):
    def body(q0, q1, msr, d0, d1, b2r, o):
        degt = d0[0, :, 0:1] + d1[0, :, 0:1] + 1.0
        dis = lax.rsqrt(degt)
        t = (q0[0, 0] + q1[0, 0] + msr[0]) * dis + b2r[...]
        mx = jnp.max(t, axis=1, keepdims=True)
        e = jnp.exp(t - mx)
        lse = jnp.log(jnp.sum(e, axis=1, keepdims=True))
        o[...] = t - mx - lse

    return pl.pallas_call(
        body,
        grid=(_G,),
        in_specs=[
            pl.BlockSpec((1, 1, _R, OUT_DIM), lambda i: (0, 0, i, 0)),
            pl.BlockSpec((1, 1, _R, OUT_DIM), lambda i: (1, 0, i, 0)),
            pl.BlockSpec((1, _R, OUT_DIM), lambda i: (0, i, 0)),
            pl.BlockSpec((1, _R, DEG_W), lambda i: (0, i, 0)),
            pl.BlockSpec((1, _R, DEG_W), lambda i: (1, i, 0)),
            pl.BlockSpec((1, OUT_DIM), lambda i: (0, 0)),
        ],
        out_specs=pl.BlockSpec((_R, OUT_DIM), lambda i: (i, 0)),
        out_shape=jax.ShapeDtypeStruct((N_NODES, OUT_DIM), jnp.float32),
    )(q, q, ms, deg, deg, b2.reshape(1, OUT_DIM))


def kernel(x, edge_index, W1, b1, W2, b2):
    src = edge_index[0].astype(jnp.int32)
    dst = edge_index[1].astype(jnp.int32)

    n_pad = EPW_PAD - EPW
    pad_src = (jnp.arange(n_pad, dtype=jnp.int32) * 89) % N_NODES
    pad_dst = N_NODES + (jnp.arange(n_pad, dtype=jnp.int32) % (NR - N_NODES))
    srcp = jnp.concatenate(
        [src.reshape(NW, EPW), jnp.broadcast_to(pad_src, (NW, n_pad))], axis=1
    ).reshape(NW, CH_IDX, K)
    dstp = jnp.concatenate(
        [dst.reshape(NW, EPW), jnp.broadcast_to(pad_dst, (NW, n_pad))], axis=1
    ).reshape(NW, CH_IDX, K)

    zo = jnp.stack([jnp.zeros((K, DEG_W), jnp.float32),
                    jnp.ones((K, DEG_W), jnp.float32)])
    deg = _deg_kernel()(zo, dstp)
    xs = _tc_scale_kernel(deg, x)
    p = _scatter_add_kernel(2)(xs, srcp, dstp)
    ms = _tc_mid_kernel(p, xs, deg, W1, b1, W2)
    q = _scatter_add_kernel(1)(ms, srcp, dstp)
    return _tc_final_kernel(q, ms, deg, b2)


# speedup vs baseline: 34.4391x; 1.0749x over previous
import functools

import jax
import jax.numpy as jnp
from jax import lax
from jax.experimental import pallas as pl
from jax.experimental.pallas import tpu as pltpu
from jax.experimental.pallas import tpu_sc as plsc

N_NODES = 10000
N_EDGES = 320000
IN_DIM = 128
HID_DIM = 256
OUT_DIM = 64

NC = 2
NS = 16
NW = NC * NS
K = 128
EPW = N_EDGES // NW
CH = 80
CH_IDX = CH
EPW_PAD = CH_IDX * K
NR = 10112
STRIPE = NR // NS
DEG_W = 4


FW = 64


def _scatter_add_kernel(nph):
    mesh = plsc.VectorSubcoreMesh(core_axis_name="c", subcore_axis_name="s")

    @functools.partial(
        pl.kernel,
        out_type=jax.ShapeDtypeStruct((NC, nph, N_NODES, FW), jnp.float32),
        mesh=mesh,
        compiler_params=pltpu.CompilerParams(use_tc_tiling_on_sc=False),
        scratch_types=[
            pltpu.VMEM((CH_IDX, K), jnp.int32),
            pltpu.VMEM((CH_IDX, K), jnp.int32),
            pltpu.VMEM((4, K, FW), jnp.float32),
            pltpu.VMEM_SHARED((NR, FW), jnp.float32),
            pltpu.SemaphoreType.DMA,
            pltpu.SemaphoreType.DMA,
            pltpu.SemaphoreType.DMA,
            pltpu.SemaphoreType.DMA,
            pltpu.SemaphoreType.DMA,
            pltpu.SemaphoreType.DMA,
            pltpu.SemaphoreType.DMA,
            pltpu.SemaphoreType.DMA,
        ],
    )
    def k(table_hbm, srcp_hbm, dstp_hbm, out_hbm, src_v, dst_v, rows_v, acc,
          g0, g1, g2, g3, s0, s1, s2, s3):
        cid = lax.axis_index("c")
        sid = lax.axis_index("s")
        wid = cid * NS + sid

        pltpu.sync_copy(srcp_hbm.at[wid], src_v)
        pltpu.sync_copy(dstp_hbm.at[wid], dst_v)

        z = jnp.zeros((16,), jnp.float32)

        def zrow(i, _):
            for c in range(FW // 16):
                rows_v[0, i, pl.ds(c * 16, 16)] = z
            return 0

        lax.fori_loop(0, K, zrow, 0)
        base = sid * STRIPE
        gsems = (g0, g1, g2, g3)
        ssems = (s0, s1, s2, s3)
        last = N_NODES - (NS - 1) * STRIPE

        for ph in range(nph):
            if ph:
                lax.fori_loop(0, K, zrow, 0)
            n_full = STRIPE // K
            for t in range(n_full):
                pltpu.sync_copy(rows_v.at[0], acc.at[pl.ds(base + t * K, K)])
            rem = STRIPE - n_full * K
            if rem:
                pltpu.sync_copy(rows_v.at[0, pl.ds(0, rem)],
                                acc.at[pl.ds(base + n_full * K, rem)])
            plsc.subcore_barrier()

            tab = table_hbm.at[ph]
            for b in range(4):
                pltpu.async_copy(tab.at[src_v.at[b]], rows_v.at[b], gsems[b])
            for b in range(4):
                pltpu.make_async_copy(
                    tab.at[src_v.at[b]], rows_v.at[b], gsems[b]).wait()
                pltpu.async_copy(rows_v.at[b], acc.at[dst_v.at[b]], ssems[b],
                                 add=True)

            def body(i, _):
                for b in range(4):
                    j = 4 * i + b
                    pltpu.make_async_copy(
                        rows_v.at[b], acc.at[dst_v.at[j - 4]],
                        ssems[b]).wait()
                    pltpu.async_copy(tab.at[src_v.at[j]], rows_v.at[b],
                                     gsems[b])
                for b in range(4):
                    j = 4 * i + b
                    pltpu.make_async_copy(
                        tab.at[src_v.at[j]], rows_v.at[b], gsems[b]).wait()
                    pltpu.async_copy(rows_v.at[b], acc.at[dst_v.at[j]],
                                     ssems[b], add=True)
                return 0

            lax.fori_loop(1, CH // 4, body, 0)
            for b in range(4):
                pltpu.make_async_copy(
                    rows_v.at[b], acc.at[dst_v.at[CH - 4 + b]],
                    ssems[b]).wait()
            plsc.subcore_barrier()

            @pl.when(sid < NS - 1)
            def _():
                pltpu.sync_copy(acc.at[pl.ds(base, STRIPE)],
                                out_hbm.at[cid, ph, pl.ds(base, STRIPE)])

            @pl.when(sid == NS - 1)
            def _():
                pltpu.sync_copy(
                    acc.at[pl.ds((NS - 1) * STRIPE, last)],
                    out_hbm.at[cid, ph, pl.ds((NS - 1) * STRIPE, last)])

            if ph + 1 < nph:
                plsc.subcore_barrier()

    return k


def _deg_kernel():
    mesh = plsc.VectorSubcoreMesh(core_axis_name="c", subcore_axis_name="s")

    @functools.partial(
        pl.kernel,
        out_type=jax.ShapeDtypeStruct((NC, N_NODES, DEG_W), jnp.float32),
        mesh=mesh,
        scratch_types=[
            pltpu.VMEM((CH_IDX, K), jnp.int32),
            pltpu.VMEM((2, K, DEG_W), jnp.float32),
            pltpu.VMEM_SHARED((NR, DEG_W), jnp.float32),
        ],
    )
    def k(zo_hbm, dstp_hbm, out_hbm, dst_v, zo_v, acc):
        cid = lax.axis_index("c")
        sid = lax.axis_index("s")
        wid = cid * NS + sid

        pltpu.sync_copy(dstp_hbm.at[wid], dst_v)
        pltpu.sync_copy(zo_hbm, zo_v)
        ones_v = zo_v.at[1]

        base = sid * STRIPE
        n_full = STRIPE // K
        for t in range(n_full):
            pltpu.sync_copy(zo_v.at[0], acc.at[pl.ds(base + t * K, K)])
        rem = STRIPE - n_full * K
        if rem:
            pltpu.sync_copy(zo_v.at[0, pl.ds(0, rem)],
                            acc.at[pl.ds(base + n_full * K, rem)])
        plsc.subcore_barrier()

        def body(j, _):
            pltpu.sync_copy(ones_v, acc.at[dst_v.at[j]], add=True)
            return 0

        lax.fori_loop(0, CH, body, 0)
        plsc.subcore_barrier()

        @pl.when(sid < NS - 1)
        def _():
            pltpu.sync_copy(acc.at[pl.ds(base, STRIPE)],
                            out_hbm.at[cid, pl.ds(base, STRIPE)])

        @pl.when(sid == NS - 1)
        def _():
            last = N_NODES - (NS - 1) * STRIPE
            pltpu.sync_copy(acc.at[pl.ds((NS - 1) * STRIPE, last)],
                            out_hbm.at[cid, pl.ds((NS - 1) * STRIPE, last)])

    return k


_R = 1000
_G = N_NODES // _R


def _tc_scale_kernel(deg, x):
    def body(d0, d1, xr, o):
        degt = d0[0, :, 0:1] + d1[0, :, 0:1] + 1.0
        dis = lax.rsqrt(degt)
        o[0] = xr[:, :FW] * dis
        o[1] = xr[:, FW:] * dis

    return pl.pallas_call(
        body,
        grid=(_G,),
        in_specs=[
            pl.BlockSpec((1, _R, DEG_W), lambda i: (0, i, 0)),
            pl.BlockSpec((1, _R, DEG_W), lambda i: (1, i, 0)),
            pl.BlockSpec((_R, IN_DIM), lambda i: (i, 0)),
        ],
        out_specs=pl.BlockSpec((2, _R, FW), lambda i: (0, i, 0)),
        out_shape=jax.ShapeDtypeStruct((2, N_NODES, FW), jnp.float32),
    )(deg, deg, x)


def _tc_mid_kernel(p, xs, deg, W1, b1, W2):
    def body(p0, p1, xsr, d0, d1, w1, b1r, w2, o):
        degt = d0[0, :, 0:1] + d1[0, :, 0:1] + 1.0
        dis = lax.rsqrt(degt)
        pr = jnp.concatenate(
            [p0[0, 0] + p1[0, 0] + xsr[0], p0[0, 1] + p1[0, 1] + xsr[1]],
            axis=1) * dis
        h = jnp.maximum(
            jnp.dot(pr, w1[...], preferred_element_type=jnp.float32) + b1r[...],
            0.0)
        m = jnp.dot(h, w2[...], preferred_element_type=jnp.float32)
        o[0] = m * dis

    return pl.pallas_call(
        body,
        grid=(_G,),
        in_specs=[
            pl.BlockSpec((1, 2, _R, FW), lambda i: (0, 0, i, 0)),
            pl.BlockSpec((1, 2, _R, FW), lambda i: (1, 0, i, 0)),
            pl.BlockSpec((2, _R, FW), lambda i: (0, i, 0)),
            pl.BlockSpec((1, _R, DEG_W), lambda i: (0, i, 0)),
            pl.BlockSpec((1, _R, DEG_W), lambda i: (1, i, 0)),
            pl.BlockSpec((IN_DIM, HID_DIM), lambda i: (0, 0)),
            pl.BlockSpec((1, HID_DIM), lambda i: (0, 0)),
            pl.BlockSpec((HID_DIM, OUT_DIM), lambda i: (0, 0)),
        ],
        out_specs=pl.BlockSpec((1, _R, OUT_DIM), lambda i: (0, i, 0)),
        out_shape=jax.ShapeDtypeStruct((1, N_NODES, OUT_DIM), jnp.float32),
    )(p, p, xs, deg, deg, W1, b1.reshape(1, HID_DIM), W2)


def _tc_final_kernel(q, ms, deg, b2):
    def body(q0, q1, msr, d0, d1, b2r, o):
        degt = d0[0, :, 0:1] + d1[0, :, 0:1] + 1.0
        dis = lax.rsqrt(degt)
        t = (q0[0, 0] + q1[0, 0] + msr[0]) * dis + b2r[...]
        mx = jnp.max(t, axis=1, keepdims=True)
        e = jnp.exp(t - mx)
        lse = jnp.log(jnp.sum(e, axis=1, keepdims=True))
        o[...] = t - mx - lse

    return pl.pallas_call(
        body,
        grid=(_G,),
        in_specs=[
            pl.BlockSpec((1, 1, _R, OUT_DIM), lambda i: (0, 0, i, 0)),
            pl.BlockSpec((1, 1, _R, OUT_DIM), lambda i: (1, 0, i, 0)),
            pl.BlockSpec((1, _R, OUT_DIM), lambda i: (0, i, 0)),
            pl.BlockSpec((1, _R, DEG_W), lambda i: (0, i, 0)),
            pl.BlockSpec((1, _R, DEG_W), lambda i: (1, i, 0)),
            pl.BlockSpec((1, OUT_DIM), lambda i: (0, 0)),
        ],
        out_specs=pl.BlockSpec((_R, OUT_DIM), lambda i: (i, 0)),
        out_shape=jax.ShapeDtypeStruct((N_NODES, OUT_DIM), jnp.float32),
    )(q, q, ms, deg, deg, b2.reshape(1, OUT_DIM))


def kernel(x, edge_index, W1, b1, W2, b2):
    src = edge_index[0].astype(jnp.int32)
    dst = edge_index[1].astype(jnp.int32)

    n_pad = EPW_PAD - EPW
    pad_src = (jnp.arange(n_pad, dtype=jnp.int32) * 89) % N_NODES
    pad_dst = N_NODES + (jnp.arange(n_pad, dtype=jnp.int32) % (NR - N_NODES))
    srcp = jnp.concatenate(
        [src.reshape(NW, EPW), jnp.broadcast_to(pad_src, (NW, n_pad))], axis=1
    ).reshape(NW, CH_IDX, K)
    dstp = jnp.concatenate(
        [dst.reshape(NW, EPW), jnp.broadcast_to(pad_dst, (NW, n_pad))], axis=1
    ).reshape(NW, CH_IDX, K)

    zo = jnp.stack([jnp.zeros((K, DEG_W), jnp.float32),
                    jnp.ones((K, DEG_W), jnp.float32)])
    deg = _deg_kernel()(zo, dstp)
    xs = _tc_scale_kernel(deg, x)
    p = _scatter_add_kernel(2)(xs, srcp, dstp)
    ms = _tc_mid_kernel(p, xs, deg, W1, b1, W2)
    q = _scatter_add_kernel(1)(ms, srcp, dstp)
    return _tc_final_kernel(q, ms, deg, b2)


# speedup vs baseline: 36.2211x; 1.0517x over previous
import functools

import jax
import jax.numpy as jnp
from jax import lax
from jax.experimental import pallas as pl
from jax.experimental.pallas import tpu as pltpu
from jax.experimental.pallas import tpu_sc as plsc

N_NODES = 10000
N_EDGES = 320000
IN_DIM = 128
HID_DIM = 256
OUT_DIM = 64

NC = 2
NS = 16
NW = NC * NS
K = 128
EPW = N_EDGES // NW
CH = 80
CH_IDX = CH
EPW_PAD = CH_IDX * K
NR = 10112
STRIPE = NR // NS
DEG_W = 4


FW = 64
CH1 = 2 * CH


def _pass1_kernel():
    mesh = plsc.VectorSubcoreMesh(core_axis_name="c", subcore_axis_name="s")

    @functools.partial(
        pl.kernel,
        out_type=jax.ShapeDtypeStruct((NC, N_NODES, FW), jnp.float32),
        mesh=mesh,
        compiler_params=pltpu.CompilerParams(use_tc_tiling_on_sc=False),
        scratch_types=[
            pltpu.VMEM((CH1, K), jnp.int32),
            pltpu.VMEM((CH1, K), jnp.int32),
            pltpu.VMEM((4, K, FW), jnp.float32),
            pltpu.VMEM_SHARED((NR, FW), jnp.float32),
            pltpu.SemaphoreType.DMA,
            pltpu.SemaphoreType.DMA,
            pltpu.SemaphoreType.DMA,
            pltpu.SemaphoreType.DMA,
            pltpu.SemaphoreType.DMA,
            pltpu.SemaphoreType.DMA,
            pltpu.SemaphoreType.DMA,
            pltpu.SemaphoreType.DMA,
        ],
    )
    def k(table_hbm, srcp_hbm, dstp_hbm, out_hbm, src_v, dst_v, rows_v, acc,
          g0, g1, g2, g3, s0, s1, s2, s3):
        cid = lax.axis_index("c")
        sid = lax.axis_index("s")

        pltpu.sync_copy(srcp_hbm.at[sid], src_v)
        pltpu.sync_copy(dstp_hbm.at[sid], dst_v)

        z = jnp.zeros((16,), jnp.float32)

        def zrow(i, _):
            for c in range(FW // 16):
                rows_v[0, i, pl.ds(c * 16, 16)] = z
            return 0

        lax.fori_loop(0, K, zrow, 0)
        base = sid * STRIPE
        gsems = (g0, g1, g2, g3)
        ssems = (s0, s1, s2, s3)
        last = N_NODES - (NS - 1) * STRIPE

        n_full = STRIPE // K
        for t in range(n_full):
            pltpu.sync_copy(rows_v.at[0], acc.at[pl.ds(base + t * K, K)])
        rem = STRIPE - n_full * K
        if rem:
            pltpu.sync_copy(rows_v.at[0, pl.ds(0, rem)],
                            acc.at[pl.ds(base + n_full * K, rem)])
        plsc.subcore_barrier()

        tab = table_hbm.at[cid]
        for b in range(4):
            pltpu.async_copy(tab.at[src_v.at[b]], rows_v.at[b], gsems[b])
        for b in range(4):
            pltpu.make_async_copy(
                tab.at[src_v.at[b]], rows_v.at[b], gsems[b]).wait()
            pltpu.async_copy(rows_v.at[b], acc.at[dst_v.at[b]], ssems[b],
                             add=True)

        def body(i, _):
            for b in range(4):
                j = 4 * i + b
                pltpu.make_async_copy(
                    rows_v.at[b], acc.at[dst_v.at[j - 4]], ssems[b]).wait()
                pltpu.async_copy(tab.at[src_v.at[j]], rows_v.at[b], gsems[b])
            for b in range(4):
                j = 4 * i + b
                pltpu.make_async_copy(
                    tab.at[src_v.at[j]], rows_v.at[b], gsems[b]).wait()
                pltpu.async_copy(rows_v.at[b], acc.at[dst_v.at[j]], ssems[b],
                                 add=True)
            return 0

        lax.fori_loop(1, CH1 // 4, body, 0)
        for b in range(4):
            pltpu.make_async_copy(
                rows_v.at[b], acc.at[dst_v.at[CH1 - 4 + b]], ssems[b]).wait()
        plsc.subcore_barrier()

        @pl.when(sid < NS - 1)
        def _():
            pltpu.sync_copy(acc.at[pl.ds(base, STRIPE)],
                            out_hbm.at[cid, pl.ds(base, STRIPE)])

        @pl.when(sid == NS - 1)
        def _():
            pltpu.sync_copy(acc.at[pl.ds((NS - 1) * STRIPE, last)],
                            out_hbm.at[cid, pl.ds((NS - 1) * STRIPE, last)])

    return k


def _scatter_add_kernel(nph):
    mesh = plsc.VectorSubcoreMesh(core_axis_name="c", subcore_axis_name="s")

    @functools.partial(
        pl.kernel,
        out_type=jax.ShapeDtypeStruct((NC, nph, N_NODES, FW), jnp.float32),
        mesh=mesh,
        compiler_params=pltpu.CompilerParams(use_tc_tiling_on_sc=False),
        scratch_types=[
            pltpu.VMEM((CH_IDX, K), jnp.int32),
            pltpu.VMEM((CH_IDX, K), jnp.int32),
            pltpu.VMEM((4, K, FW), jnp.float32),
            pltpu.VMEM_SHARED((NR, FW), jnp.float32),
            pltpu.SemaphoreType.DMA,
            pltpu.SemaphoreType.DMA,
            pltpu.SemaphoreType.DMA,
            pltpu.SemaphoreType.DMA,
            pltpu.SemaphoreType.DMA,
            pltpu.SemaphoreType.DMA,
            pltpu.SemaphoreType.DMA,
            pltpu.SemaphoreType.DMA,
        ],
    )
    def k(table_hbm, srcp_hbm, dstp_hbm, out_hbm, src_v, dst_v, rows_v, acc,
          g0, g1, g2, g3, s0, s1, s2, s3):
        cid = lax.axis_index("c")
        sid = lax.axis_index("s")
        wid = cid * NS + sid

        pltpu.sync_copy(srcp_hbm.at[wid], src_v)
        pltpu.sync_copy(dstp_hbm.at[wid], dst_v)

        z = jnp.zeros((16,), jnp.float32)

        def zrow(i, _):
            for c in range(FW // 16):
                rows_v[0, i, pl.ds(c * 16, 16)] = z
            return 0

        lax.fori_loop(0, K, zrow, 0)
        base = sid * STRIPE
        gsems = (g0, g1, g2, g3)
        ssems = (s0, s1, s2, s3)
        last = N_NODES - (NS - 1) * STRIPE

        for ph in range(nph):
            if ph:
                lax.fori_loop(0, K, zrow, 0)
            n_full = STRIPE // K
            for t in range(n_full):
                pltpu.sync_copy(rows_v.at[0], acc.at[pl.ds(base + t * K, K)])
            rem = STRIPE - n_full * K
            if rem:
                pltpu.sync_copy(rows_v.at[0, pl.ds(0, rem)],
                                acc.at[pl.ds(base + n_full * K, rem)])
            plsc.subcore_barrier()

            tab = table_hbm.at[ph]
            for b in range(4):
                pltpu.async_copy(tab.at[src_v.at[b]], rows_v.at[b], gsems[b])
            for b in range(4):
                pltpu.make_async_copy(
                    tab.at[src_v.at[b]], rows_v.at[b], gsems[b]).wait()
                pltpu.async_copy(rows_v.at[b], acc.at[dst_v.at[b]], ssems[b],
                                 add=True)

            def body(i, _):
                for b in range(4):
                    j = 4 * i + b
                    pltpu.make_async_copy(
                        rows_v.at[b], acc.at[dst_v.at[j - 4]],
                        ssems[b]).wait()
                    pltpu.async_copy(tab.at[src_v.at[j]], rows_v.at[b],
                                     gsems[b])
                for b in range(4):
                    j = 4 * i + b
                    pltpu.make_async_copy(
                        tab.at[src_v.at[j]], rows_v.at[b], gsems[b]).wait()
                    pltpu.async_copy(rows_v.at[b], acc.at[dst_v.at[j]],
                                     ssems[b], add=True)
                return 0

            lax.fori_loop(1, CH // 4, body, 0)
            for b in range(4):
                pltpu.make_async_copy(
                    rows_v.at[b], acc.at[dst_v.at[CH - 4 + b]],
                    ssems[b]).wait()
            plsc.subcore_barrier()

            @pl.when(sid < NS - 1)
            def _():
                pltpu.sync_copy(acc.at[pl.ds(base, STRIPE)],
                                out_hbm.at[cid, ph, pl.ds(base, STRIPE)])

            @pl.when(sid == NS - 1)
            def _():
                pltpu.sync_copy(
                    acc.at[pl.ds((NS - 1) * STRIPE, last)],
                    out_hbm.at[cid, ph, pl.ds((NS - 1) * STRIPE, last)])

            if ph + 1 < nph:
                plsc.subcore_barrier()

    return k


def _deg_kernel():
    mesh = plsc.VectorSubcoreMesh(core_axis_name="c", subcore_axis_name="s")

    @functools.partial(
        pl.kernel,
        out_type=jax.ShapeDtypeStruct((NC, N_NODES, DEG_W), jnp.float32),
        mesh=mesh,
        scratch_types=[
            pltpu.VMEM((CH_IDX, K), jnp.int32),
            pltpu.VMEM((2, K, DEG_W), jnp.float32),
            pltpu.VMEM_SHARED((NR, DEG_W), jnp.float32),
        ],
    )
    def k(zo_hbm, dstp_hbm, out_hbm, dst_v, zo_v, acc):
        cid = lax.axis_index("c")
        sid = lax.axis_index("s")
        wid = cid * NS + sid

        pltpu.sync_copy(dstp_hbm.at[wid], dst_v)
        pltpu.sync_copy(zo_hbm, zo_v)
        ones_v = zo_v.at[1]

        base = sid * STRIPE
        n_full = STRIPE // K
        for t in range(n_full):
            pltpu.sync_copy(zo_v.at[0], acc.at[pl.ds(base + t * K, K)])
        rem = STRIPE - n_full * K
        if rem:
            pltpu.sync_copy(zo_v.at[0, pl.ds(0, rem)],
                            acc.at[pl.ds(base + n_full * K, rem)])
        plsc.subcore_barrier()

        def body(j, _):
            pltpu.sync_copy(ones_v, acc.at[dst_v.at[j]], add=True)
            return 0

        lax.fori_loop(0, CH, body, 0)
        plsc.subcore_barrier()

        @pl.when(sid < NS - 1)
        def _():
            pltpu.sync_copy(acc.at[pl.ds(base, STRIPE)],
                            out_hbm.at[cid, pl.ds(base, STRIPE)])

        @pl.when(sid == NS - 1)
        def _():
            last = N_NODES - (NS - 1) * STRIPE
            pltpu.sync_copy(acc.at[pl.ds((NS - 1) * STRIPE, last)],
                            out_hbm.at[cid, pl.ds((NS - 1) * STRIPE, last)])

    return k


_R = 1000
_G = N_NODES // _R


def _tc_scale_kernel(deg, x):
    def body(d0, d1, xr, o):
        degt = d0[0, :, 0:1] + d1[0, :, 0:1] + 1.0
        dis = lax.rsqrt(degt)
        o[0] = xr[:, :FW] * dis
        o[1] = xr[:, FW:] * dis

    return pl.pallas_call(
        body,
        grid=(_G,),
        in_specs=[
            pl.BlockSpec((1, _R, DEG_W), lambda i: (0, i, 0)),
            pl.BlockSpec((1, _R, DEG_W), lambda i: (1, i, 0)),
            pl.BlockSpec((_R, IN_DIM), lambda i: (i, 0)),
        ],
        out_specs=pl.BlockSpec((2, _R, FW), lambda i: (0, i, 0)),
        out_shape=jax.ShapeDtypeStruct((2, N_NODES, FW), jnp.float32),
    )(deg, deg, x)


def _tc_mid_kernel(p, xs, deg, W1, b1, W2):
    def body(p0, p1, xsr, d0, d1, w1, b1r, w2, o):
        degt = d0[0, :, 0:1] + d1[0, :, 0:1] + 1.0
        dis = lax.rsqrt(degt)
        pr = jnp.concatenate(
            [p0[0] + xsr[0], p1[0] + xsr[1]], axis=1) * dis
        h = jnp.maximum(
            jnp.dot(pr, w1[...], preferred_element_type=jnp.float32) + b1r[...],
            0.0)
        m = jnp.dot(h, w2[...], preferred_element_type=jnp.float32)
        o[0] = m * dis

    return pl.pallas_call(
        body,
        grid=(_G,),
        in_specs=[
            pl.BlockSpec((1, _R, FW), lambda i: (0, i, 0)),
            pl.BlockSpec((1, _R, FW), lambda i: (1, i, 0)),
            pl.BlockSpec((2, _R, FW), lambda i: (0, i, 0)),
            pl.BlockSpec((1, _R, DEG_W), lambda i: (0, i, 0)),
            pl.BlockSpec((1, _R, DEG_W), lambda i: (1, i, 0)),
            pl.BlockSpec((IN_DIM, HID_DIM), lambda i: (0, 0)),
            pl.BlockSpec((1, HID_DIM), lambda i: (0, 0)),
            pl.BlockSpec((HID_DIM, OUT_DIM), lambda i: (0, 0)),
        ],
        out_specs=pl.BlockSpec((1, _R, OUT_DIM), lambda i: (0, i, 0)),
        out_shape=jax.ShapeDtypeStruct((1, N_NODES, OUT_DIM), jnp.float32),
    )(p, p, xs, deg, deg, W1, b1.reshape(1, HID_DIM), W2)


def _tc_final_kernel(q, ms, deg, b2):
    def body(q0, q1, msr, d0, d1, b2r, o):
        degt = d0[0, :, 0:1] + d1[0, :, 0:1] + 1.0
        dis = lax.rsqrt(degt)
        t = (q0[0, 0] + q1[0, 0] + msr[0]) * dis + b2r[...]
        mx = jnp.max(t, axis=1, keepdims=True)
        e = jnp.exp(t - mx)
        lse = jnp.log(jnp.sum(e, axis=1, keepdims=True))
        o[...] = t - mx - lse

    return pl.pallas_call(
        body,
        grid=(_G,),
        in_specs=[
            pl.BlockSpec((1, 1, _R, OUT_DIM), lambda i: (0, 0, i, 0)),
            pl.BlockSpec((1, 1, _R, OUT_DIM), lambda i: (1, 0, i, 0)),
            pl.BlockSpec((1, _R, OUT_DIM), lambda i: (0, i, 0)),
            pl.BlockSpec((1, _R, DEG_W), lambda i: (0, i, 0)),
            pl.BlockSpec((1, _R, DEG_W), lambda i: (1, i, 0)),
            pl.BlockSpec((1, OUT_DIM), lambda i: (0, 0)),
        ],
        out_specs=pl.BlockSpec((_R, OUT_DIM), lambda i: (i, 0)),
        out_shape=jax.ShapeDtypeStruct((N_NODES, OUT_DIM), jnp.float32),
    )(q, q, ms, deg, deg, b2.reshape(1, OUT_DIM))


def kernel(x, edge_index, W1, b1, W2, b2):
    src = edge_index[0].astype(jnp.int32)
    dst = edge_index[1].astype(jnp.int32)

    n_pad = EPW_PAD - EPW
    pad_src = (jnp.arange(n_pad, dtype=jnp.int32) * 89) % N_NODES
    pad_dst = N_NODES + (jnp.arange(n_pad, dtype=jnp.int32) % (NR - N_NODES))
    srcp = jnp.concatenate(
        [src.reshape(NW, EPW), jnp.broadcast_to(pad_src, (NW, n_pad))], axis=1
    ).reshape(NW, CH_IDX, K)
    dstp = jnp.concatenate(
        [dst.reshape(NW, EPW), jnp.broadcast_to(pad_dst, (NW, n_pad))], axis=1
    ).reshape(NW, CH_IDX, K)

    zo = jnp.stack([jnp.zeros((K, DEG_W), jnp.float32),
                    jnp.ones((K, DEG_W), jnp.float32)])
    deg = _deg_kernel()(zo, dstp)
    xs = _tc_scale_kernel(deg, x)
    srcp1 = srcp.reshape(NS, CH1, K)
    dstp1 = dstp.reshape(NS, CH1, K)
    p = _pass1_kernel()(xs, srcp1, dstp1)
    ms = _tc_mid_kernel(p, xs, deg, W1, b1, W2)
    q = _scatter_add_kernel(1)(ms, srcp, dstp)
    return _tc_final_kernel(q, ms, deg, b2)


# speedup vs baseline: 36.8497x; 1.0174x over previous
import functools

import jax
import jax.numpy as jnp
from jax import lax
from jax.experimental import pallas as pl
from jax.experimental.pallas import tpu as pltpu
from jax.experimental.pallas import tpu_sc as plsc

N_NODES = 10000
N_EDGES = 320000
IN_DIM = 128
HID_DIM = 256
OUT_DIM = 64

NC = 2
NS = 16
NW = NC * NS
K = 128
EPW = N_EDGES // NW
CH = 80
CH_IDX = CH
EPW_PAD = CH_IDX * K
NR = 10112
STRIPE = NR // NS
DEG_W = 4


FW = 64
CH1 = 2 * CH


def _pass1_kernel():
    mesh = plsc.VectorSubcoreMesh(core_axis_name="c", subcore_axis_name="s")

    @functools.partial(
        pl.kernel,
        out_type=jax.ShapeDtypeStruct((NC, N_NODES, FW), jnp.float32),
        mesh=mesh,
        compiler_params=pltpu.CompilerParams(use_tc_tiling_on_sc=False),
        scratch_types=[
            pltpu.VMEM((CH1, K), jnp.int32),
            pltpu.VMEM((CH1, K), jnp.int32),
            pltpu.VMEM((4, K, FW), jnp.float32),
            pltpu.VMEM_SHARED((NR, FW), jnp.float32),
            pltpu.SemaphoreType.DMA,
            pltpu.SemaphoreType.DMA,
            pltpu.SemaphoreType.DMA,
            pltpu.SemaphoreType.DMA,
            pltpu.SemaphoreType.DMA,
            pltpu.SemaphoreType.DMA,
            pltpu.SemaphoreType.DMA,
            pltpu.SemaphoreType.DMA,
        ],
    )
    def k(table_hbm, srcp_hbm, dstp_hbm, out_hbm, src_v, dst_v, rows_v, acc,
          g0, g1, g2, g3, s0, s1, s2, s3):
        cid = lax.axis_index("c")
        sid = lax.axis_index("s")

        pltpu.sync_copy(srcp_hbm.at[sid], src_v)
        pltpu.sync_copy(dstp_hbm.at[sid], dst_v)

        z = jnp.zeros((16,), jnp.float32)

        def zrow(i, _):
            for c in range(FW // 16):
                rows_v[0, i, pl.ds(c * 16, 16)] = z
            return 0

        lax.fori_loop(0, K, zrow, 0)
        base = sid * STRIPE
        gsems = (g0, g1, g2, g3)
        ssems = (s0, s1, s2, s3)
        last = N_NODES - (NS - 1) * STRIPE

        n_full = STRIPE // K
        for t in range(n_full):
            pltpu.sync_copy(rows_v.at[0], acc.at[pl.ds(base + t * K, K)])
        rem = STRIPE - n_full * K
        if rem:
            pltpu.sync_copy(rows_v.at[0, pl.ds(0, rem)],
                            acc.at[pl.ds(base + n_full * K, rem)])
        plsc.subcore_barrier()

        tab = table_hbm.at[cid]
        for b in range(4):
            pltpu.async_copy(tab.at[src_v.at[b]], rows_v.at[b], gsems[b])
        for b in range(4):
            pltpu.make_async_copy(
                tab.at[src_v.at[b]], rows_v.at[b], gsems[b]).wait()
            pltpu.async_copy(rows_v.at[b], acc.at[dst_v.at[b]], ssems[b],
                             add=True)

        def body(i, _):
            for b in range(4):
                j = 4 * i + b
                pltpu.make_async_copy(
                    rows_v.at[b], acc.at[dst_v.at[j - 4]], ssems[b]).wait()
                pltpu.async_copy(tab.at[src_v.at[j]], rows_v.at[b], gsems[b])
            for b in range(4):
                j = 4 * i + b
                pltpu.make_async_copy(
                    tab.at[src_v.at[j]], rows_v.at[b], gsems[b]).wait()
                pltpu.async_copy(rows_v.at[b], acc.at[dst_v.at[j]], ssems[b],
                                 add=True)
            return 0

        lax.fori_loop(1, CH1 // 4, body, 0)
        for b in range(4):
            pltpu.make_async_copy(
                rows_v.at[b], acc.at[dst_v.at[CH1 - 4 + b]], ssems[b]).wait()
        plsc.subcore_barrier()

        @pl.when(sid < NS - 1)
        def _():
            pltpu.sync_copy(acc.at[pl.ds(base, STRIPE)],
                            out_hbm.at[cid, pl.ds(base, STRIPE)])

        @pl.when(sid == NS - 1)
        def _():
            pltpu.sync_copy(acc.at[pl.ds((NS - 1) * STRIPE, last)],
                            out_hbm.at[cid, pl.ds((NS - 1) * STRIPE, last)])

    return k


def _scatter_add_kernel(nph):
    mesh = plsc.VectorSubcoreMesh(core_axis_name="c", subcore_axis_name="s")

    @functools.partial(
        pl.kernel,
        out_type=jax.ShapeDtypeStruct((NC, nph, N_NODES, FW), jnp.float32),
        mesh=mesh,
        compiler_params=pltpu.CompilerParams(use_tc_tiling_on_sc=False),
        scratch_types=[
            pltpu.VMEM((CH_IDX, K), jnp.int32),
            pltpu.VMEM((CH_IDX, K), jnp.int32),
            pltpu.VMEM((4, K, FW), jnp.float32),
            pltpu.VMEM_SHARED((NR, FW), jnp.float32),
            pltpu.SemaphoreType.DMA,
            pltpu.SemaphoreType.DMA,
            pltpu.SemaphoreType.DMA,
            pltpu.SemaphoreType.DMA,
            pltpu.SemaphoreType.DMA,
            pltpu.SemaphoreType.DMA,
            pltpu.SemaphoreType.DMA,
            pltpu.SemaphoreType.DMA,
        ],
    )
    def k(table_hbm, srcp_hbm, dstp_hbm, out_hbm, src_v, dst_v, rows_v, acc,
          g0, g1, g2, g3, s0, s1, s2, s3):
        cid = lax.axis_index("c")
        sid = lax.axis_index("s")
        wid = cid * NS + sid

        pltpu.sync_copy(srcp_hbm.at[wid], src_v)
        pltpu.sync_copy(dstp_hbm.at[wid], dst_v)

        z = jnp.zeros((16,), jnp.float32)

        def zrow(i, _):
            for c in range(FW // 16):
                rows_v[0, i, pl.ds(c * 16, 16)] = z
            return 0

        lax.fori_loop(0, K, zrow, 0)
        base = sid * STRIPE
        gsems = (g0, g1, g2, g3)
        ssems = (s0, s1, s2, s3)
        last = N_NODES - (NS - 1) * STRIPE

        for ph in range(nph):
            if ph:
                lax.fori_loop(0, K, zrow, 0)
            n_full = STRIPE // K
            for t in range(n_full):
                pltpu.sync_copy(rows_v.at[0], acc.at[pl.ds(base + t * K, K)])
            rem = STRIPE - n_full * K
            if rem:
                pltpu.sync_copy(rows_v.at[0, pl.ds(0, rem)],
                                acc.at[pl.ds(base + n_full * K, rem)])
            plsc.subcore_barrier()

            tab = table_hbm.at[ph]
            for b in range(4):
                pltpu.async_copy(tab.at[src_v.at[b]], rows_v.at[b], gsems[b])
            for b in range(4):
                pltpu.make_async_copy(
                    tab.at[src_v.at[b]], rows_v.at[b], gsems[b]).wait()
                pltpu.async_copy(rows_v.at[b], acc.at[dst_v.at[b]], ssems[b],
                                 add=True)

            def body(i, _):
                for b in range(4):
                    j = 4 * i + b
                    pltpu.make_async_copy(
                        rows_v.at[b], acc.at[dst_v.at[j - 4]],
                        ssems[b]).wait()
                    pltpu.async_copy(tab.at[src_v.at[j]], rows_v.at[b],
                                     gsems[b])
                for b in range(4):
                    j = 4 * i + b
                    pltpu.make_async_copy(
                        tab.at[src_v.at[j]], rows_v.at[b], gsems[b]).wait()
                    pltpu.async_copy(rows_v.at[b], acc.at[dst_v.at[j]],
                                     ssems[b], add=True)
                return 0

            lax.fori_loop(1, CH // 4, body, 0)
            for b in range(4):
                pltpu.make_async_copy(
                    rows_v.at[b], acc.at[dst_v.at[CH - 4 + b]],
                    ssems[b]).wait()
            plsc.subcore_barrier()

            @pl.when(sid < NS - 1)
            def _():
                pltpu.sync_copy(acc.at[pl.ds(base, STRIPE)],
                                out_hbm.at[cid, ph, pl.ds(base, STRIPE)])

            @pl.when(sid == NS - 1)
            def _():
                pltpu.sync_copy(
                    acc.at[pl.ds((NS - 1) * STRIPE, last)],
                    out_hbm.at[cid, ph, pl.ds((NS - 1) * STRIPE, last)])

            if ph + 1 < nph:
                plsc.subcore_barrier()

    return k


def _deg_kernel():
    mesh = plsc.VectorSubcoreMesh(core_axis_name="c", subcore_axis_name="s")

    @functools.partial(
        pl.kernel,
        out_type=jax.ShapeDtypeStruct((NC, N_NODES, DEG_W), jnp.float32),
        mesh=mesh,
        scratch_types=[
            pltpu.VMEM((CH_IDX, K), jnp.int32),
            pltpu.VMEM((2, K, DEG_W), jnp.float32),
            pltpu.VMEM_SHARED((NR, DEG_W), jnp.float32),
            pltpu.SemaphoreType.DMA,
        ],
    )
    def k(zo_hbm, dstp_hbm, out_hbm, dst_v, zo_v, acc, sem):
        cid = lax.axis_index("c")
        sid = lax.axis_index("s")
        wid = cid * NS + sid

        pltpu.sync_copy(dstp_hbm.at[wid], dst_v)
        pltpu.sync_copy(zo_hbm, zo_v)
        ones_v = zo_v.at[1]

        base = sid * STRIPE
        n_full = STRIPE // K
        for t in range(n_full):
            pltpu.sync_copy(zo_v.at[0], acc.at[pl.ds(base + t * K, K)])
        rem = STRIPE - n_full * K
        if rem:
            pltpu.sync_copy(zo_v.at[0, pl.ds(0, rem)],
                            acc.at[pl.ds(base + n_full * K, rem)])
        plsc.subcore_barrier()

        def body(j, _):
            pltpu.async_copy(ones_v, acc.at[dst_v.at[j]], sem, add=True)
            return 0

        lax.fori_loop(0, CH, body, 0)

        def drain(j, _):
            pltpu.make_async_copy(ones_v, acc.at[dst_v.at[j]], sem).wait()
            return 0

        lax.fori_loop(0, CH, drain, 0)
        plsc.subcore_barrier()

        @pl.when(sid < NS - 1)
        def _():
            pltpu.sync_copy(acc.at[pl.ds(base, STRIPE)],
                            out_hbm.at[cid, pl.ds(base, STRIPE)])

        @pl.when(sid == NS - 1)
        def _():
            last = N_NODES - (NS - 1) * STRIPE
            pltpu.sync_copy(acc.at[pl.ds((NS - 1) * STRIPE, last)],
                            out_hbm.at[cid, pl.ds((NS - 1) * STRIPE, last)])

    return k


_R = 1000
_G = N_NODES // _R


def _tc_scale_kernel(deg, x):
    def body(d0, d1, xr, o):
        degt = d0[0, :, 0:1] + d1[0, :, 0:1] + 1.0
        dis = lax.rsqrt(degt)
        o[0] = xr[:, :FW] * dis
        o[1] = xr[:, FW:] * dis

    return pl.pallas_call(
        body,
        grid=(_G,),
        in_specs=[
            pl.BlockSpec((1, _R, DEG_W), lambda i: (0, i, 0)),
            pl.BlockSpec((1, _R, DEG_W), lambda i: (1, i, 0)),
            pl.BlockSpec((_R, IN_DIM), lambda i: (i, 0)),
        ],
        out_specs=pl.BlockSpec((2, _R, FW), lambda i: (0, i, 0)),
        out_shape=jax.ShapeDtypeStruct((2, N_NODES, FW), jnp.float32),
    )(deg, deg, x)


def _tc_mid_kernel(p, xs, deg, W1, b1, W2):
    def body(p0, p1, xsr, d0, d1, w1, b1r, w2, o):
        degt = d0[0, :, 0:1] + d1[0, :, 0:1] + 1.0
        dis = lax.rsqrt(degt)
        pr = jnp.concatenate(
            [p0[0] + xsr[0], p1[0] + xsr[1]], axis=1) * dis
        h = jnp.maximum(
            jnp.dot(pr, w1[...], preferred_element_type=jnp.float32) + b1r[...],
            0.0)
        m = jnp.dot(h, w2[...], preferred_element_type=jnp.float32)
        o[0] = m * dis

    return pl.pallas_call(
        body,
        grid=(_G,),
        in_specs=[
            pl.BlockSpec((1, _R, FW), lambda i: (0, i, 0)),
            pl.BlockSpec((1, _R, FW), lambda i: (1, i, 0)),
            pl.BlockSpec((2, _R, FW), lambda i: (0, i, 0)),
            pl.BlockSpec((1, _R, DEG_W), lambda i: (0, i, 0)),
            pl.BlockSpec((1, _R, DEG_W), lambda i: (1, i, 0)),
            pl.BlockSpec((IN_DIM, HID_DIM), lambda i: (0, 0)),
            pl.BlockSpec((1, HID_DIM), lambda i: (0, 0)),
            pl.BlockSpec((HID_DIM, OUT_DIM), lambda i: (0, 0)),
        ],
        out_specs=pl.BlockSpec((1, _R, OUT_DIM), lambda i: (0, i, 0)),
        out_shape=jax.ShapeDtypeStruct((1, N_NODES, OUT_DIM), jnp.float32),
    )(p, p, xs, deg, deg, W1, b1.reshape(1, HID_DIM), W2)


def _tc_final_kernel(q, ms, deg, b2):
    def body(q0, q1, msr, d0, d1, b2r, o):
        degt = d0[0, :, 0:1] + d1[0, :, 0:1] + 1.0
        dis = lax.rsqrt(degt)
        t = (q0[0, 0] + q1[0, 0] + msr[0]) * dis + b2r[...]
        mx = jnp.max(t, axis=1, keepdims=True)
        e = jnp.exp(t - mx)
        lse = jnp.log(jnp.sum(e, axis=1, keepdims=True))
        o[...] = t - mx - lse

    return pl.pallas_call(
        body,
        grid=(_G,),
        in_specs=[
            pl.BlockSpec((1, 1, _R, OUT_DIM), lambda i: (0, 0, i, 0)),
            pl.BlockSpec((1, 1, _R, OUT_DIM), lambda i: (1, 0, i, 0)),
            pl.BlockSpec((1, _R, OUT_DIM), lambda i: (0, i, 0)),
            pl.BlockSpec((1, _R, DEG_W), lambda i: (0, i, 0)),
            pl.BlockSpec((1, _R, DEG_W), lambda i: (1, i, 0)),
            pl.BlockSpec((1, OUT_DIM), lambda i: (0, 0)),
        ],
        out_specs=pl.BlockSpec((_R, OUT_DIM), lambda i: (i, 0)),
        out_shape=jax.ShapeDtypeStruct((N_NODES, OUT_DIM), jnp.float32),
    )(q, q, ms, deg, deg, b2.reshape(1, OUT_DIM))


def kernel(x, edge_index, W1, b1, W2, b2):
    src = edge_index[0].astype(jnp.int32)
    dst = edge_index[1].astype(jnp.int32)

    n_pad = EPW_PAD - EPW
    pad_src = (jnp.arange(n_pad, dtype=jnp.int32) * 89) % N_NODES
    pad_dst = N_NODES + (jnp.arange(n_pad, dtype=jnp.int32) % (NR - N_NODES))
    srcp = jnp.concatenate(
        [src.reshape(NW, EPW), jnp.broadcast_to(pad_src, (NW, n_pad))], axis=1
    ).reshape(NW, CH_IDX, K)
    dstp = jnp.concatenate(
        [dst.reshape(NW, EPW), jnp.broadcast_to(pad_dst, (NW, n_pad))], axis=1
    ).reshape(NW, CH_IDX, K)

    zo = jnp.stack([jnp.zeros((K, DEG_W), jnp.float32),
                    jnp.ones((K, DEG_W), jnp.float32)])
    deg = _deg_kernel()(zo, dstp)
    xs = _tc_scale_kernel(deg, x)
    srcp1 = srcp.reshape(NS, CH1, K)
    dstp1 = dstp.reshape(NS, CH1, K)
    p = _pass1_kernel()(xs, srcp1, dstp1)
    ms = _tc_mid_kernel(p, xs, deg, W1, b1, W2)
    q = _scatter_add_kernel(1)(ms, srcp, dstp)
    return _tc_final_kernel(q, ms, deg, b2)


# speedup vs baseline: 36.9510x; 1.0027x over previous
import functools

import jax
import jax.numpy as jnp
from jax import lax
from jax.experimental import pallas as pl
from jax.experimental.pallas import tpu as pltpu
from jax.experimental.pallas import tpu_sc as plsc

N_NODES = 10000
N_EDGES = 320000
IN_DIM = 128
HID_DIM = 256
OUT_DIM = 64

NC = 2
NS = 16
NW = NC * NS
K = 128
EPW = N_EDGES // NW
CH = 80
CH_IDX = CH
EPW_PAD = CH_IDX * K
NR = 10112
STRIPE = NR // NS
DEG_W = 4


FW = 64
CH1 = 2 * CH
NB = 4


def _pass1_kernel():
    mesh = plsc.VectorSubcoreMesh(core_axis_name="c", subcore_axis_name="s")

    @functools.partial(
        pl.kernel,
        out_type=jax.ShapeDtypeStruct((NC, N_NODES, FW), jnp.float32),
        mesh=mesh,
        compiler_params=pltpu.CompilerParams(use_tc_tiling_on_sc=False),
        scratch_types=[
            pltpu.VMEM((CH1, K), jnp.int32),
            pltpu.VMEM((CH1, K), jnp.int32),
            pltpu.VMEM((NB, K, FW), jnp.float32),
            pltpu.VMEM_SHARED((NR, FW), jnp.float32),
            pltpu.SemaphoreType.DMA,
            pltpu.SemaphoreType.DMA,
            pltpu.SemaphoreType.DMA,
            pltpu.SemaphoreType.DMA,
            pltpu.SemaphoreType.DMA,
            pltpu.SemaphoreType.DMA,
            pltpu.SemaphoreType.DMA,
            pltpu.SemaphoreType.DMA,
        ],
    )
    def k(table_hbm, srcp_hbm, dstp_hbm, out_hbm, src_v, dst_v, rows_v, acc,
          *sems):
        cid = lax.axis_index("c")
        sid = lax.axis_index("s")

        pltpu.sync_copy(srcp_hbm.at[sid], src_v)
        pltpu.sync_copy(dstp_hbm.at[sid], dst_v)

        z = jnp.zeros((16,), jnp.float32)

        def zrow(i, _):
            for c in range(FW // 16):
                rows_v[0, i, pl.ds(c * 16, 16)] = z
            return 0

        lax.fori_loop(0, K, zrow, 0)
        base = sid * STRIPE
        gsems = sems[:NB]
        ssems = sems[NB:2 * NB]
        last = N_NODES - (NS - 1) * STRIPE

        n_full = STRIPE // K
        for t in range(n_full):
            pltpu.sync_copy(rows_v.at[0], acc.at[pl.ds(base + t * K, K)])
        rem = STRIPE - n_full * K
        if rem:
            pltpu.sync_copy(rows_v.at[0, pl.ds(0, rem)],
                            acc.at[pl.ds(base + n_full * K, rem)])
        plsc.subcore_barrier()

        tab = table_hbm.at[cid]
        for b in range(NB):
            pltpu.async_copy(tab.at[src_v.at[b]], rows_v.at[b], gsems[b])
        for b in range(NB):
            pltpu.make_async_copy(
                tab.at[src_v.at[b]], rows_v.at[b], gsems[b]).wait()
            pltpu.async_copy(rows_v.at[b], acc.at[dst_v.at[b]], ssems[b],
                             add=True)

        def body(i, _):
            for b in range(NB):
                j = NB * i + b
                pltpu.make_async_copy(
                    rows_v.at[b], acc.at[dst_v.at[j - NB]], ssems[b]).wait()
                pltpu.async_copy(tab.at[src_v.at[j]], rows_v.at[b], gsems[b])
            for b in range(NB):
                j = NB * i + b
                pltpu.make_async_copy(
                    tab.at[src_v.at[j]], rows_v.at[b], gsems[b]).wait()
                pltpu.async_copy(rows_v.at[b], acc.at[dst_v.at[j]], ssems[b],
                                 add=True)
            return 0

        lax.fori_loop(1, CH1 // NB, body, 0)
        for b in range(NB):
            pltpu.make_async_copy(
                rows_v.at[b], acc.at[dst_v.at[CH1 - NB + b]], ssems[b]).wait()
        plsc.subcore_barrier()

        @pl.when(sid < NS - 1)
        def _():
            pltpu.sync_copy(acc.at[pl.ds(base, STRIPE)],
                            out_hbm.at[cid, pl.ds(base, STRIPE)])

        @pl.when(sid == NS - 1)
        def _():
            pltpu.sync_copy(acc.at[pl.ds((NS - 1) * STRIPE, last)],
                            out_hbm.at[cid, pl.ds((NS - 1) * STRIPE, last)])

    return k


def _scatter_add_kernel(nph):
    mesh = plsc.VectorSubcoreMesh(core_axis_name="c", subcore_axis_name="s")

    @functools.partial(
        pl.kernel,
        out_type=jax.ShapeDtypeStruct((NC, nph, N_NODES, FW), jnp.float32),
        mesh=mesh,
        compiler_params=pltpu.CompilerParams(use_tc_tiling_on_sc=False),
        scratch_types=[
            pltpu.VMEM((CH_IDX, K), jnp.int32),
            pltpu.VMEM((CH_IDX, K), jnp.int32),
            pltpu.VMEM((NB, K, FW), jnp.float32),
            pltpu.VMEM_SHARED((NR, FW), jnp.float32),
            pltpu.SemaphoreType.DMA,
            pltpu.SemaphoreType.DMA,
            pltpu.SemaphoreType.DMA,
            pltpu.SemaphoreType.DMA,
            pltpu.SemaphoreType.DMA,
            pltpu.SemaphoreType.DMA,
            pltpu.SemaphoreType.DMA,
            pltpu.SemaphoreType.DMA,
        ],
    )
    def k(table_hbm, srcp_hbm, dstp_hbm, out_hbm, src_v, dst_v, rows_v, acc,
          *sems):
        cid = lax.axis_index("c")
        sid = lax.axis_index("s")
        wid = cid * NS + sid

        pltpu.sync_copy(srcp_hbm.at[wid], src_v)
        pltpu.sync_copy(dstp_hbm.at[wid], dst_v)

        z = jnp.zeros((16,), jnp.float32)

        def zrow(i, _):
            for c in range(FW // 16):
                rows_v[0, i, pl.ds(c * 16, 16)] = z
            return 0

        lax.fori_loop(0, K, zrow, 0)
        base = sid * STRIPE
        gsems = sems[:NB]
        ssems = sems[NB:2 * NB]
        last = N_NODES - (NS - 1) * STRIPE

        for ph in range(nph):
            if ph:
                lax.fori_loop(0, K, zrow, 0)
            n_full = STRIPE // K
            for t in range(n_full):
                pltpu.sync_copy(rows_v.at[0], acc.at[pl.ds(base + t * K, K)])
            rem = STRIPE - n_full * K
            if rem:
                pltpu.sync_copy(rows_v.at[0, pl.ds(0, rem)],
                                acc.at[pl.ds(base + n_full * K, rem)])
            plsc.subcore_barrier()

            tab = table_hbm.at[ph]
            for b in range(NB):
                pltpu.async_copy(tab.at[src_v.at[b]], rows_v.at[b], gsems[b])
            for b in range(NB):
                pltpu.make_async_copy(
                    tab.at[src_v.at[b]], rows_v.at[b], gsems[b]).wait()
                pltpu.async_copy(rows_v.at[b], acc.at[dst_v.at[b]], ssems[b],
                                 add=True)

            def body(i, _):
                for b in range(NB):
                    j = NB * i + b
                    pltpu.make_async_copy(
                        rows_v.at[b], acc.at[dst_v.at[j - NB]],
                        ssems[b]).wait()
                    pltpu.async_copy(tab.at[src_v.at[j]], rows_v.at[b],
                                     gsems[b])
                for b in range(NB):
                    j = NB * i + b
                    pltpu.make_async_copy(
                        tab.at[src_v.at[j]], rows_v.at[b], gsems[b]).wait()
                    pltpu.async_copy(rows_v.at[b], acc.at[dst_v.at[j]],
                                     ssems[b], add=True)
                return 0

            lax.fori_loop(1, CH // NB, body, 0)
            for b in range(NB):
                pltpu.make_async_copy(
                    rows_v.at[b], acc.at[dst_v.at[CH - NB + b]],
                    ssems[b]).wait()
            plsc.subcore_barrier()

            @pl.when(sid < NS - 1)
            def _():
                pltpu.sync_copy(acc.at[pl.ds(base, STRIPE)],
                                out_hbm.at[cid, ph, pl.ds(base, STRIPE)])

            @pl.when(sid == NS - 1)
            def _():
                pltpu.sync_copy(
                    acc.at[pl.ds((NS - 1) * STRIPE, last)],
                    out_hbm.at[cid, ph, pl.ds((NS - 1) * STRIPE, last)])

            if ph + 1 < nph:
                plsc.subcore_barrier()

    return k


def _deg_kernel():
    mesh = plsc.VectorSubcoreMesh(core_axis_name="c", subcore_axis_name="s")

    @functools.partial(
        pl.kernel,
        out_type=jax.ShapeDtypeStruct((NC, N_NODES, DEG_W), jnp.float32),
        mesh=mesh,
        scratch_types=[
            pltpu.VMEM((CH_IDX, K), jnp.int32),
            pltpu.VMEM((2, K, DEG_W), jnp.float32),
            pltpu.VMEM_SHARED((NR, DEG_W), jnp.float32),
            pltpu.SemaphoreType.DMA,
        ],
    )
    def k(zo_hbm, dstp_hbm, out_hbm, dst_v, zo_v, acc, sem):
        cid = lax.axis_index("c")
        sid = lax.axis_index("s")
        wid = cid * NS + sid

        pltpu.sync_copy(dstp_hbm.at[wid], dst_v)
        pltpu.sync_copy(zo_hbm, zo_v)
        ones_v = zo_v.at[1]

        base = sid * STRIPE
        n_full = STRIPE // K
        for t in range(n_full):
            pltpu.sync_copy(zo_v.at[0], acc.at[pl.ds(base + t * K, K)])
        rem = STRIPE - n_full * K
        if rem:
            pltpu.sync_copy(zo_v.at[0, pl.ds(0, rem)],
                            acc.at[pl.ds(base + n_full * K, rem)])
        plsc.subcore_barrier()

        def body(j, _):
            pltpu.async_copy(ones_v, acc.at[dst_v.at[j]], sem, add=True)
            return 0

        lax.fori_loop(0, CH, body, 0)

        def drain(j, _):
            pltpu.make_async_copy(ones_v, acc.at[dst_v.at[j]], sem).wait()
            return 0

        lax.fori_loop(0, CH, drain, 0)
        plsc.subcore_barrier()

        @pl.when(sid < NS - 1)
        def _():
            pltpu.sync_copy(acc.at[pl.ds(base, STRIPE)],
                            out_hbm.at[cid, pl.ds(base, STRIPE)])

        @pl.when(sid == NS - 1)
        def _():
            last = N_NODES - (NS - 1) * STRIPE
            pltpu.sync_copy(acc.at[pl.ds((NS - 1) * STRIPE, last)],
                            out_hbm.at[cid, pl.ds((NS - 1) * STRIPE, last)])

    return k


_R = 1000
_G = N_NODES // _R


def _tc_scale_kernel(deg, x):
    def body(d0, d1, xr, o):
        degt = d0[0, :, 0:1] + d1[0, :, 0:1] + 1.0
        dis = lax.rsqrt(degt)
        o[0] = xr[:, :FW] * dis
        o[1] = xr[:, FW:] * dis

    return pl.pallas_call(
        body,
        grid=(_G,),
        in_specs=[
            pl.BlockSpec((1, _R, DEG_W), lambda i: (0, i, 0)),
            pl.BlockSpec((1, _R, DEG_W), lambda i: (1, i, 0)),
            pl.BlockSpec((_R, IN_DIM), lambda i: (i, 0)),
        ],
        out_specs=pl.BlockSpec((2, _R, FW), lambda i: (0, i, 0)),
        out_shape=jax.ShapeDtypeStruct((2, N_NODES, FW), jnp.float32),
    )(deg, deg, x)


def _tc_mid_kernel(p, xs, deg, W1, b1, W2):
    def body(p0, p1, xsr, d0, d1, w1, b1r, w2, o):
        degt = d0[0, :, 0:1] + d1[0, :, 0:1] + 1.0
        dis = lax.rsqrt(degt)
        pr = jnp.concatenate(
            [p0[0] + xsr[0], p1[0] + xsr[1]], axis=1) * dis
        h = jnp.maximum(
            jnp.dot(pr, w1[...], preferred_element_type=jnp.float32) + b1r[...],
            0.0)
        m = jnp.dot(h, w2[...], preferred_element_type=jnp.float32)
        o[0] = m * dis

    return pl.pallas_call(
        body,
        grid=(_G,),
        in_specs=[
            pl.BlockSpec((1, _R, FW), lambda i: (0, i, 0)),
            pl.BlockSpec((1, _R, FW), lambda i: (1, i, 0)),
            pl.BlockSpec((2, _R, FW), lambda i: (0, i, 0)),
            pl.BlockSpec((1, _R, DEG_W), lambda i: (0, i, 0)),
            pl.BlockSpec((1, _R, DEG_W), lambda i: (1, i, 0)),
            pl.BlockSpec((IN_DIM, HID_DIM), lambda i: (0, 0)),
            pl.BlockSpec((1, HID_DIM), lambda i: (0, 0)),
            pl.BlockSpec((HID_DIM, OUT_DIM), lambda i: (0, 0)),
        ],
        out_specs=pl.BlockSpec((1, _R, OUT_DIM), lambda i: (0, i, 0)),
        out_shape=jax.ShapeDtypeStruct((1, N_NODES, OUT_DIM), jnp.float32),
    )(p, p, xs, deg, deg, W1, b1.reshape(1, HID_DIM), W2)


def _tc_final_kernel(q, ms, deg, b2):
    def body(q0, q1, msr, d0, d1, b2r, o):
        degt = d0[0, :, 0:1] + d1[0, :, 0:1] + 1.0
        dis = lax.rsqrt(degt)
        t = (q0[0, 0] + q1[0, 0] + msr[0]) * dis + b2r[...]
        mx = jnp.max(t, axis=1, keepdims=True)
        e = jnp.exp(t - mx)
        lse = jnp.log(jnp.sum(e, axis=1, keepdims=True))
        o[...] = t - mx - lse

    return pl.pallas_call(
        body,
        grid=(_G,),
        in_specs=[
            pl.BlockSpec((1, 1, _R, OUT_DIM), lambda i: (0, 0, i, 0)),
            pl.BlockSpec((1, 1, _R, OUT_DIM), lambda i: (1, 0, i, 0)),
            pl.BlockSpec((1, _R, OUT_DIM), lambda i: (0, i, 0)),
            pl.BlockSpec((1, _R, DEG_W), lambda i: (0, i, 0)),
            pl.BlockSpec((1, _R, DEG_W), lambda i: (1, i, 0)),
            pl.BlockSpec((1, OUT_DIM), lambda i: (0, 0)),
        ],
        out_specs=pl.BlockSpec((_R, OUT_DIM), lambda i: (i, 0)),
        out_shape=jax.ShapeDtypeStruct((N_NODES, OUT_DIM), jnp.float32),
    )(q, q, ms, deg, deg, b2.reshape(1, OUT_DIM))


def kernel(x, edge_index, W1, b1, W2, b2):
    src = edge_index[0].astype(jnp.int32)
    dst = edge_index[1].astype(jnp.int32)

    n_pad = EPW_PAD - EPW
    pad_src = (jnp.arange(n_pad, dtype=jnp.int32) * 89) % N_NODES
    pad_dst = N_NODES + (jnp.arange(n_pad, dtype=jnp.int32) % (NR - N_NODES))
    srcp = jnp.concatenate(
        [src.reshape(NW, EPW), jnp.broadcast_to(pad_src, (NW, n_pad))], axis=1
    ).reshape(NW, CH_IDX, K)
    dstp = jnp.concatenate(
        [dst.reshape(NW, EPW), jnp.broadcast_to(pad_dst, (NW, n_pad))], axis=1
    ).reshape(NW, CH_IDX, K)

    zo = jnp.stack([jnp.zeros((K, DEG_W), jnp.float32),
                    jnp.ones((K, DEG_W), jnp.float32)])
    deg = _deg_kernel()(zo, dstp)
    xs = _tc_scale_kernel(deg, x)
    srcp1 = srcp.reshape(NS, CH1, K)
    dstp1 = dstp.reshape(NS, CH1, K)
    p = _pass1_kernel()(xs, srcp1, dstp1)
    ms = _tc_mid_kernel(p, xs, deg, W1, b1, W2)
    q = _scatter_add_kernel(1)(ms, srcp, dstp)
    return _tc_final_kernel(q, ms, deg, b2)


# speedup vs baseline: 37.6144x; 1.0180x over previous
import functools

import jax
import jax.numpy as jnp
from jax import lax
from jax.experimental import pallas as pl
from jax.experimental.pallas import tpu as pltpu
from jax.experimental.pallas import tpu_sc as plsc

N_NODES = 10000
N_EDGES = 320000
IN_DIM = 128
HID_DIM = 256
OUT_DIM = 64

NC = 2
NS = 16
NW = NC * NS
K = 128
EPW = N_EDGES // NW
CH = 80
CH_IDX = CH
EPW_PAD = CH_IDX * K
NR = 10112
STRIPE = NR // NS
DEG_W = 4


FW = 64
CH1 = 2 * CH
NB = 4


def _pass1_kernel():
    mesh = plsc.VectorSubcoreMesh(core_axis_name="c", subcore_axis_name="s")

    @functools.partial(
        pl.kernel,
        out_type=jax.ShapeDtypeStruct((NC, N_NODES, FW), jnp.float32),
        mesh=mesh,
        compiler_params=pltpu.CompilerParams(use_tc_tiling_on_sc=False),
        scratch_types=[
            pltpu.VMEM((CH1, K), jnp.int32),
            pltpu.VMEM((CH1, K), jnp.int32),
            pltpu.VMEM((NB, K, FW), jnp.float32),
            pltpu.VMEM_SHARED((NR, FW), jnp.float32),
            pltpu.SemaphoreType.DMA,
            pltpu.SemaphoreType.DMA,
            pltpu.SemaphoreType.DMA,
            pltpu.SemaphoreType.DMA,
            pltpu.SemaphoreType.DMA,
            pltpu.SemaphoreType.DMA,
            pltpu.SemaphoreType.DMA,
            pltpu.SemaphoreType.DMA,
        ],
    )
    def k(table_hbm, srcp_hbm, dstp_hbm, out_hbm, src_v, dst_v, rows_v, acc,
          *sems):
        cid = lax.axis_index("c")
        sid = lax.axis_index("s")

        pltpu.sync_copy(srcp_hbm.at[sid], src_v)
        pltpu.sync_copy(dstp_hbm.at[sid], dst_v)

        z = jnp.zeros((16,), jnp.float32)

        def zrow(i, _):
            for c in range(FW // 16):
                rows_v[0, i, pl.ds(c * 16, 16)] = z
            return 0

        lax.fori_loop(0, K, zrow, 0)
        base = sid * STRIPE
        gsems = sems[:NB]
        ssems = sems[NB:2 * NB]
        last = N_NODES - (NS - 1) * STRIPE

        n_full = STRIPE // K
        for t in range(n_full):
            pltpu.sync_copy(rows_v.at[0], acc.at[pl.ds(base + t * K, K)])
        rem = STRIPE - n_full * K
        if rem:
            pltpu.sync_copy(rows_v.at[0, pl.ds(0, rem)],
                            acc.at[pl.ds(base + n_full * K, rem)])
        plsc.subcore_barrier()

        tab = table_hbm.at[cid]
        for b in range(NB):
            pltpu.async_copy(tab.at[src_v.at[b]], rows_v.at[b], gsems[b])
        for b in range(NB):
            pltpu.make_async_copy(
                tab.at[src_v.at[b]], rows_v.at[b], gsems[b]).wait()
            pltpu.async_copy(rows_v.at[b], acc.at[dst_v.at[b]], ssems[b],
                             add=True)

        def body(i, _):
            for b in range(NB):
                j = NB * i + b
                pltpu.make_async_copy(
                    rows_v.at[b], acc.at[dst_v.at[j - NB]], ssems[b]).wait()
                pltpu.async_copy(tab.at[src_v.at[j]], rows_v.at[b], gsems[b])
            for b in range(NB):
                j = NB * i + b
                pltpu.make_async_copy(
                    tab.at[src_v.at[j]], rows_v.at[b], gsems[b]).wait()
                pltpu.async_copy(rows_v.at[b], acc.at[dst_v.at[j]], ssems[b],
                                 add=True)
            return 0

        lax.fori_loop(1, CH1 // NB, body, 0)
        for b in range(NB):
            pltpu.make_async_copy(
                rows_v.at[b], acc.at[dst_v.at[CH1 - NB + b]], ssems[b]).wait()
        plsc.subcore_barrier()

        @pl.when(sid < NS - 1)
        def _():
            pltpu.sync_copy(acc.at[pl.ds(base, STRIPE)],
                            out_hbm.at[cid, pl.ds(base, STRIPE)])

        @pl.when(sid == NS - 1)
        def _():
            pltpu.sync_copy(acc.at[pl.ds((NS - 1) * STRIPE, last)],
                            out_hbm.at[cid, pl.ds((NS - 1) * STRIPE, last)])

    return k


def _scatter_add_kernel(nph):
    mesh = plsc.VectorSubcoreMesh(core_axis_name="c", subcore_axis_name="s")

    @functools.partial(
        pl.kernel,
        out_type=jax.ShapeDtypeStruct((NC, nph, N_NODES, FW), jnp.float32),
        mesh=mesh,
        compiler_params=pltpu.CompilerParams(use_tc_tiling_on_sc=False),
        scratch_types=[
            pltpu.VMEM((CH_IDX, K), jnp.int32),
            pltpu.VMEM((CH_IDX, K), jnp.int32),
            pltpu.VMEM((NB, K, FW), jnp.float32),
            pltpu.VMEM_SHARED((NR, FW), jnp.float32),
            pltpu.SemaphoreType.DMA,
            pltpu.SemaphoreType.DMA,
            pltpu.SemaphoreType.DMA,
            pltpu.SemaphoreType.DMA,
            pltpu.SemaphoreType.DMA,
            pltpu.SemaphoreType.DMA,
            pltpu.SemaphoreType.DMA,
            pltpu.SemaphoreType.DMA,
        ],
    )
    def k(table_hbm, srcp_hbm, dstp_hbm, out_hbm, src_v, dst_v, rows_v, acc,
          *sems):
        cid = lax.axis_index("c")
        sid = lax.axis_index("s")
        wid = cid * NS + sid

        pltpu.sync_copy(srcp_hbm.at[wid], src_v)
        pltpu.sync_copy(dstp_hbm.at[wid], dst_v)

        z = jnp.zeros((16,), jnp.float32)

        def zrow(i, _):
            for c in range(FW // 16):
                rows_v[0, i, pl.ds(c * 16, 16)] = z
            return 0

        lax.fori_loop(0, K, zrow, 0)
        base = sid * STRIPE
        gsems = sems[:NB]
        ssems = sems[NB:2 * NB]
        last = N_NODES - (NS - 1) * STRIPE

        for ph in range(nph):
            if ph:
                lax.fori_loop(0, K, zrow, 0)
            n_full = STRIPE // K
            for t in range(n_full):
                pltpu.sync_copy(rows_v.at[0], acc.at[pl.ds(base + t * K, K)])
            rem = STRIPE - n_full * K
            if rem:
                pltpu.sync_copy(rows_v.at[0, pl.ds(0, rem)],
                                acc.at[pl.ds(base + n_full * K, rem)])
            plsc.subcore_barrier()

            tab = table_hbm.at[ph]
            for b in range(NB):
                pltpu.async_copy(tab.at[src_v.at[b]], rows_v.at[b], gsems[b])
            for b in range(NB):
                pltpu.make_async_copy(
                    tab.at[src_v.at[b]], rows_v.at[b], gsems[b]).wait()
                pltpu.async_copy(rows_v.at[b], acc.at[dst_v.at[b]], ssems[b],
                                 add=True)

            def body(i, _):
                for b in range(NB):
                    j = NB * i + b
                    pltpu.make_async_copy(
                        rows_v.at[b], acc.at[dst_v.at[j - NB]],
                        ssems[b]).wait()
                    pltpu.async_copy(tab.at[src_v.at[j]], rows_v.at[b],
                                     gsems[b])
                for b in range(NB):
                    j = NB * i + b
                    pltpu.make_async_copy(
                        tab.at[src_v.at[j]], rows_v.at[b], gsems[b]).wait()
                    pltpu.async_copy(rows_v.at[b], acc.at[dst_v.at[j]],
                                     ssems[b], add=True)
                return 0

            lax.fori_loop(1, CH // NB, body, 0)
            for b in range(NB):
                pltpu.make_async_copy(
                    rows_v.at[b], acc.at[dst_v.at[CH - NB + b]],
                    ssems[b]).wait()
            plsc.subcore_barrier()

            @pl.when(sid < NS - 1)
            def _():
                pltpu.sync_copy(acc.at[pl.ds(base, STRIPE)],
                                out_hbm.at[cid, ph, pl.ds(base, STRIPE)])

            @pl.when(sid == NS - 1)
            def _():
                pltpu.sync_copy(
                    acc.at[pl.ds((NS - 1) * STRIPE, last)],
                    out_hbm.at[cid, ph, pl.ds((NS - 1) * STRIPE, last)])

            if ph + 1 < nph:
                plsc.subcore_barrier()

    return k


def _deg_kernel():
    mesh = plsc.VectorSubcoreMesh(core_axis_name="c", subcore_axis_name="s")

    @functools.partial(
        pl.kernel,
        out_type=jax.ShapeDtypeStruct((NC, N_NODES, DEG_W), jnp.float32),
        mesh=mesh,
        scratch_types=[
            pltpu.VMEM((CH_IDX, K), jnp.int32),
            pltpu.VMEM((2, K, DEG_W), jnp.float32),
            pltpu.VMEM_SHARED((NR, DEG_W), jnp.float32),
            pltpu.SemaphoreType.DMA,
        ],
    )
    def k(zo_hbm, dstp_hbm, out_hbm, dst_v, zo_v, acc, sem):
        cid = lax.axis_index("c")
        sid = lax.axis_index("s")
        wid = cid * NS + sid

        pltpu.sync_copy(dstp_hbm.at[wid], dst_v)
        pltpu.sync_copy(zo_hbm, zo_v)
        ones_v = zo_v.at[1]

        base = sid * STRIPE
        n_full = STRIPE // K
        for t in range(n_full):
            pltpu.sync_copy(zo_v.at[0], acc.at[pl.ds(base + t * K, K)])
        rem = STRIPE - n_full * K
        if rem:
            pltpu.sync_copy(zo_v.at[0, pl.ds(0, rem)],
                            acc.at[pl.ds(base + n_full * K, rem)])
        plsc.subcore_barrier()

        def body(j, _):
            pltpu.async_copy(ones_v, acc.at[dst_v.at[j]], sem, add=True)
            return 0

        lax.fori_loop(0, CH, body, 0)

        def drain(j, _):
            pltpu.make_async_copy(ones_v, acc.at[dst_v.at[j]], sem).wait()
            return 0

        lax.fori_loop(0, CH, drain, 0)
        plsc.subcore_barrier()

        @pl.when(sid < NS - 1)
        def _():
            pltpu.sync_copy(acc.at[pl.ds(base, STRIPE)],
                            out_hbm.at[cid, pl.ds(base, STRIPE)])

        @pl.when(sid == NS - 1)
        def _():
            last = N_NODES - (NS - 1) * STRIPE
            pltpu.sync_copy(acc.at[pl.ds((NS - 1) * STRIPE, last)],
                            out_hbm.at[cid, pl.ds((NS - 1) * STRIPE, last)])

    return k


_R = 2000
_G = N_NODES // _R


def _tc_scale_kernel(deg, x):
    def body(d0, d1, xr, o):
        degt = d0[0, :, 0:1] + d1[0, :, 0:1] + 1.0
        dis = lax.rsqrt(degt)
        o[0] = xr[:, :FW] * dis
        o[1] = xr[:, FW:] * dis

    return pl.pallas_call(
        body,
        grid=(_G,),
        in_specs=[
            pl.BlockSpec((1, _R, DEG_W), lambda i: (0, i, 0)),
            pl.BlockSpec((1, _R, DEG_W), lambda i: (1, i, 0)),
            pl.BlockSpec((_R, IN_DIM), lambda i: (i, 0)),
        ],
        out_specs=pl.BlockSpec((2, _R, FW), lambda i: (0, i, 0)),
        out_shape=jax.ShapeDtypeStruct((2, N_NODES, FW), jnp.float32),
    )(deg, deg, x)


def _tc_mid_kernel(p, xs, deg, W1, b1, W2):
    def body(p0, p1, xsr, d0, d1, w1, b1r, w2, o):
        degt = d0[0, :, 0:1] + d1[0, :, 0:1] + 1.0
        dis = lax.rsqrt(degt)
        pr = jnp.concatenate(
            [p0[0] + xsr[0], p1[0] + xsr[1]], axis=1) * dis
        h = jnp.maximum(
            jnp.dot(pr, w1[...], preferred_element_type=jnp.float32) + b1r[...],
            0.0)
        m = jnp.dot(h, w2[...], preferred_element_type=jnp.float32)
        o[0] = m * dis

    return pl.pallas_call(
        body,
        grid=(_G,),
        in_specs=[
            pl.BlockSpec((1, _R, FW), lambda i: (0, i, 0)),
            pl.BlockSpec((1, _R, FW), lambda i: (1, i, 0)),
            pl.BlockSpec((2, _R, FW), lambda i: (0, i, 0)),
            pl.BlockSpec((1, _R, DEG_W), lambda i: (0, i, 0)),
            pl.BlockSpec((1, _R, DEG_W), lambda i: (1, i, 0)),
            pl.BlockSpec((IN_DIM, HID_DIM), lambda i: (0, 0)),
            pl.BlockSpec((1, HID_DIM), lambda i: (0, 0)),
            pl.BlockSpec((HID_DIM, OUT_DIM), lambda i: (0, 0)),
        ],
        out_specs=pl.BlockSpec((1, _R, OUT_DIM), lambda i: (0, i, 0)),
        out_shape=jax.ShapeDtypeStruct((1, N_NODES, OUT_DIM), jnp.float32),
    )(p, p, xs, deg, deg, W1, b1.reshape(1, HID_DIM), W2)


def _tc_final_kernel(q, ms, deg, b2):
    def body(q0, q1, msr, d0, d1, b2r, o):
        degt = d0[0, :, 0:1] + d1[0, :, 0:1] + 1.0
        dis = lax.rsqrt(degt)
        t = (q0[0, 0] + q1[0, 0] + msr[0]) * dis + b2r[...]
        mx = jnp.max(t, axis=1, keepdims=True)
        e = jnp.exp(t - mx)
        lse = jnp.log(jnp.sum(e, axis=1, keepdims=True))
        o[...] = t - mx - lse

    return pl.pallas_call(
        body,
        grid=(_G,),
        in_specs=[
            pl.BlockSpec((1, 1, _R, OUT_DIM), lambda i: (0, 0, i, 0)),
            pl.BlockSpec((1, 1, _R, OUT_DIM), lambda i: (1, 0, i, 0)),
            pl.BlockSpec((1, _R, OUT_DIM), lambda i: (0, i, 0)),
            pl.BlockSpec((1, _R, DEG_W), lambda i: (0, i, 0)),
            pl.BlockSpec((1, _R, DEG_W), lambda i: (1, i, 0)),
            pl.BlockSpec((1, OUT_DIM), lambda i: (0, 0)),
        ],
        out_specs=pl.BlockSpec((_R, OUT_DIM), lambda i: (i, 0)),
        out_shape=jax.ShapeDtypeStruct((N_NODES, OUT_DIM), jnp.float32),
    )(q, q, ms, deg, deg, b2.reshape(1, OUT_DIM))


def kernel(x, edge_index, W1, b1, W2, b2):
    src = edge_index[0].astype(jnp.int32)
    dst = edge_index[1].astype(jnp.int32)

    n_pad = EPW_PAD - EPW
    pad_src = (jnp.arange(n_pad, dtype=jnp.int32) * 89) % N_NODES
    pad_dst = N_NODES + (jnp.arange(n_pad, dtype=jnp.int32) % (NR - N_NODES))
    srcp = jnp.concatenate(
        [src.reshape(NW, EPW), jnp.broadcast_to(pad_src, (NW, n_pad))], axis=1
    ).reshape(NW, CH_IDX, K)
    dstp = jnp.concatenate(
        [dst.reshape(NW, EPW), jnp.broadcast_to(pad_dst, (NW, n_pad))], axis=1
    ).reshape(NW, CH_IDX, K)

    zo = jnp.stack([jnp.zeros((K, DEG_W), jnp.float32),
                    jnp.ones((K, DEG_W), jnp.float32)])
    deg = _deg_kernel()(zo, dstp)
    xs = _tc_scale_kernel(deg, x)
    srcp1 = srcp.reshape(NS, CH1, K)
    dstp1 = dstp.reshape(NS, CH1, K)
    p = _pass1_kernel()(xs, srcp1, dstp1)
    ms = _tc_mid_kernel(p, xs, deg, W1, b1, W2)
    q = _scatter_add_kernel(1)(ms, srcp, dstp)
    return _tc_final_kernel(q, ms, deg, b2)


# speedup vs baseline: 38.0658x; 1.0120x over previous
import functools

import jax
import jax.numpy as jnp
from jax import lax
from jax.experimental import pallas as pl
from jax.experimental.pallas import tpu as pltpu
from jax.experimental.pallas import tpu_sc as plsc

N_NODES = 10000
N_EDGES = 320000
IN_DIM = 128
HID_DIM = 256
OUT_DIM = 64

NC = 2
NS = 16
NW = NC * NS
K = 128
EPW = N_EDGES // NW
CH = 80
CH_IDX = CH
EPW_PAD = CH_IDX * K
NR = 10112
STRIPE = NR // NS
DEG_W = 4


TOT_CH = N_EDGES // K
FW = 64
CH1 = 2 * CH
NB = 4


def _pass1_kernel():
    mesh = plsc.VectorSubcoreMesh(core_axis_name="c", subcore_axis_name="s")

    @functools.partial(
        pl.kernel,
        out_type=jax.ShapeDtypeStruct((NC, N_NODES, FW), jnp.float32),
        mesh=mesh,
        compiler_params=pltpu.CompilerParams(use_tc_tiling_on_sc=False),
        scratch_types=[
            pltpu.VMEM((CH1, K), jnp.int32),
            pltpu.VMEM((CH1, K), jnp.int32),
            pltpu.VMEM((NB, K, FW), jnp.float32),
            pltpu.VMEM_SHARED((NR, FW), jnp.float32),
            pltpu.SemaphoreType.DMA,
            pltpu.SemaphoreType.DMA,
            pltpu.SemaphoreType.DMA,
            pltpu.SemaphoreType.DMA,
            pltpu.SemaphoreType.DMA,
            pltpu.SemaphoreType.DMA,
            pltpu.SemaphoreType.DMA,
            pltpu.SemaphoreType.DMA,
        ],
    )
    def k(table_hbm, srcp_hbm, dstp_hbm, out_hbm, src_v, dst_v, rows_v, acc,
          *sems):
        cid = lax.axis_index("c")
        sid = lax.axis_index("s")

        pltpu.sync_copy(srcp_hbm.at[sid], src_v)
        pltpu.sync_copy(dstp_hbm.at[sid], dst_v)

        z = jnp.zeros((16,), jnp.float32)

        def zrow(i, _):
            for c in range(FW // 16):
                rows_v[0, i, pl.ds(c * 16, 16)] = z
            return 0

        lax.fori_loop(0, K, zrow, 0)
        base = sid * STRIPE
        gsems = sems[:NB]
        ssems = sems[NB:2 * NB]
        last = N_NODES - (NS - 1) * STRIPE

        n_full = STRIPE // K
        for t in range(n_full):
            pltpu.sync_copy(rows_v.at[0], acc.at[pl.ds(base + t * K, K)])
        rem = STRIPE - n_full * K
        if rem:
            pltpu.sync_copy(rows_v.at[0, pl.ds(0, rem)],
                            acc.at[pl.ds(base + n_full * K, rem)])
        plsc.subcore_barrier()

        tab = table_hbm.at[cid]
        for b in range(NB):
            pltpu.async_copy(tab.at[src_v.at[b]], rows_v.at[b], gsems[b])
        for b in range(NB):
            pltpu.make_async_copy(
                tab.at[src_v.at[b]], rows_v.at[b], gsems[b]).wait()
            pltpu.async_copy(rows_v.at[b], acc.at[dst_v.at[b]], ssems[b],
                             add=True)

        def body(i, _):
            for b in range(NB):
                j = NB * i + b
                pltpu.make_async_copy(
                    rows_v.at[b], acc.at[dst_v.at[j - NB]], ssems[b]).wait()
                pltpu.async_copy(tab.at[src_v.at[j]], rows_v.at[b], gsems[b])
            for b in range(NB):
                j = NB * i + b
                pltpu.make_async_copy(
                    tab.at[src_v.at[j]], rows_v.at[b], gsems[b]).wait()
                pltpu.async_copy(rows_v.at[b], acc.at[dst_v.at[j]], ssems[b],
                                 add=True)
            return 0

        lax.fori_loop(1, CH1 // NB, body, 0)
        for b in range(NB):
            pltpu.make_async_copy(
                rows_v.at[b], acc.at[dst_v.at[CH1 - NB + b]], ssems[b]).wait()
        plsc.subcore_barrier()

        @pl.when(sid < NS - 1)
        def _():
            pltpu.sync_copy(acc.at[pl.ds(base, STRIPE)],
                            out_hbm.at[cid, pl.ds(base, STRIPE)])

        @pl.when(sid == NS - 1)
        def _():
            pltpu.sync_copy(acc.at[pl.ds((NS - 1) * STRIPE, last)],
                            out_hbm.at[cid, pl.ds((NS - 1) * STRIPE, last)])

    return k


def _scatter_add_kernel(nph):
    mesh = plsc.VectorSubcoreMesh(core_axis_name="c", subcore_axis_name="s")

    @functools.partial(
        pl.kernel,
        out_type=jax.ShapeDtypeStruct((NC, nph, N_NODES, FW), jnp.float32),
        mesh=mesh,
        compiler_params=pltpu.CompilerParams(use_tc_tiling_on_sc=False),
        scratch_types=[
            pltpu.VMEM((CH_IDX, K), jnp.int32),
            pltpu.VMEM((CH_IDX, K), jnp.int32),
            pltpu.VMEM((NB, K, FW), jnp.float32),
            pltpu.VMEM_SHARED((NR, FW), jnp.float32),
            pltpu.SemaphoreType.DMA,
            pltpu.SemaphoreType.DMA,
            pltpu.SemaphoreType.DMA,
            pltpu.SemaphoreType.DMA,
            pltpu.SemaphoreType.DMA,
            pltpu.SemaphoreType.DMA,
            pltpu.SemaphoreType.DMA,
            pltpu.SemaphoreType.DMA,
        ],
    )
    def k(table_hbm, srcp_hbm, dstp_hbm, out_hbm, src_v, dst_v, rows_v, acc,
          *sems):
        cid = lax.axis_index("c")
        sid = lax.axis_index("s")
        wid = cid * NS + sid

        pltpu.sync_copy(srcp_hbm.at[wid], src_v)
        pltpu.sync_copy(dstp_hbm.at[wid], dst_v)

        z = jnp.zeros((16,), jnp.float32)

        def zrow(i, _):
            for c in range(FW // 16):
                rows_v[0, i, pl.ds(c * 16, 16)] = z
            return 0

        lax.fori_loop(0, K, zrow, 0)
        base = sid * STRIPE
        gsems = sems[:NB]
        ssems = sems[NB:2 * NB]
        last = N_NODES - (NS - 1) * STRIPE

        for ph in range(nph):
            if ph:
                lax.fori_loop(0, K, zrow, 0)
            n_full = STRIPE // K
            for t in range(n_full):
                pltpu.sync_copy(rows_v.at[0], acc.at[pl.ds(base + t * K, K)])
            rem = STRIPE - n_full * K
            if rem:
                pltpu.sync_copy(rows_v.at[0, pl.ds(0, rem)],
                                acc.at[pl.ds(base + n_full * K, rem)])
            plsc.subcore_barrier()

            tab = table_hbm.at[ph]
            for b in range(NB):
                pltpu.async_copy(tab.at[src_v.at[b]], rows_v.at[b], gsems[b])
            for b in range(NB):
                pltpu.make_async_copy(
                    tab.at[src_v.at[b]], rows_v.at[b], gsems[b]).wait()
                pltpu.async_copy(rows_v.at[b], acc.at[dst_v.at[b]], ssems[b],
                                 add=True)

            def body(i, _):
                for b in range(NB):
                    j = NB * i + b
                    pltpu.make_async_copy(
                        rows_v.at[b], acc.at[dst_v.at[j - NB]],
                        ssems[b]).wait()
                    pltpu.async_copy(tab.at[src_v.at[j]], rows_v.at[b],
                                     gsems[b])
                for b in range(NB):
                    j = NB * i + b
                    pltpu.make_async_copy(
                        tab.at[src_v.at[j]], rows_v.at[b], gsems[b]).wait()
                    pltpu.async_copy(rows_v.at[b], acc.at[dst_v.at[j]],
                                     ssems[b], add=True)
                return 0

            lax.fori_loop(1, CH // NB, body, 0)
            for b in range(NB):
                pltpu.make_async_copy(
                    rows_v.at[b], acc.at[dst_v.at[CH - NB + b]],
                    ssems[b]).wait()
            plsc.subcore_barrier()

            @pl.when(sid < NS - 1)
            def _():
                pltpu.sync_copy(acc.at[pl.ds(base, STRIPE)],
                                out_hbm.at[cid, ph, pl.ds(base, STRIPE)])

            @pl.when(sid == NS - 1)
            def _():
                pltpu.sync_copy(
                    acc.at[pl.ds((NS - 1) * STRIPE, last)],
                    out_hbm.at[cid, ph, pl.ds((NS - 1) * STRIPE, last)])

            if ph + 1 < nph:
                plsc.subcore_barrier()

    return k


def _deg_kernel():
    mesh = plsc.VectorSubcoreMesh(core_axis_name="c", subcore_axis_name="s")

    @functools.partial(
        pl.kernel,
        out_type=jax.ShapeDtypeStruct((NC, N_NODES, DEG_W), jnp.float32),
        mesh=mesh,
        compiler_params=pltpu.CompilerParams(use_tc_tiling_on_sc=False),
        scratch_types=[
            pltpu.VMEM((TOT_CH // NW + 1, K), jnp.int32),
            pltpu.VMEM((2, K, DEG_W), jnp.float32),
            pltpu.VMEM_SHARED((NR, DEG_W), jnp.float32),
            pltpu.SemaphoreType.DMA,
        ],
    )
    def k(zo_hbm, dstr_hbm, out_hbm, dst_v, zo_v, acc, sem):
        cid = lax.axis_index("c")
        sid = lax.axis_index("s")
        wid = cid * NS + sid

        nch = TOT_CH // NW
        extra = TOT_CH - nch * NW
        cbase = wid * nch + jnp.minimum(wid, extra)
        n = jnp.where(wid < extra, nch + 1, nch)
        pltpu.sync_copy(dstr_hbm.at[pl.ds(cbase, nch)],
                        dst_v.at[pl.ds(0, nch)])

        @pl.when(wid < extra)
        def _():
            pltpu.sync_copy(dstr_hbm.at[pl.ds(cbase + nch, 1)],
                            dst_v.at[pl.ds(nch, 1)])

        pltpu.sync_copy(zo_hbm, zo_v)
        ones_v = zo_v.at[1]

        base = sid * STRIPE
        n_full = STRIPE // K
        for t in range(n_full):
            pltpu.sync_copy(zo_v.at[0], acc.at[pl.ds(base + t * K, K)])
        rem = STRIPE - n_full * K
        if rem:
            pltpu.sync_copy(zo_v.at[0, pl.ds(0, rem)],
                            acc.at[pl.ds(base + n_full * K, rem)])
        plsc.subcore_barrier()

        def body(j, _):
            pltpu.async_copy(ones_v, acc.at[dst_v.at[j]], sem, add=True)
            return 0

        lax.fori_loop(0, n, body, 0)

        def drain(j, _):
            pltpu.make_async_copy(ones_v, acc.at[dst_v.at[j]], sem).wait()
            return 0

        lax.fori_loop(0, n, drain, 0)
        plsc.subcore_barrier()

        @pl.when(sid < NS - 1)
        def _():
            pltpu.sync_copy(acc.at[pl.ds(base, STRIPE)],
                            out_hbm.at[cid, pl.ds(base, STRIPE)])

        @pl.when(sid == NS - 1)
        def _():
            last = N_NODES - (NS - 1) * STRIPE
            pltpu.sync_copy(acc.at[pl.ds((NS - 1) * STRIPE, last)],
                            out_hbm.at[cid, pl.ds((NS - 1) * STRIPE, last)])

    return k


_R = 2000
_G = N_NODES // _R


def _tc_scale_kernel(deg, x):
    def body(d0, d1, xr, o):
        degt = d0[0, :, 0:1] + d1[0, :, 0:1] + 1.0
        dis = lax.rsqrt(degt)
        o[0] = xr[:, :FW] * dis
        o[1] = xr[:, FW:] * dis

    return pl.pallas_call(
        body,
        grid=(_G,),
        in_specs=[
            pl.BlockSpec((1, _R, DEG_W), lambda i: (0, i, 0)),
            pl.BlockSpec((1, _R, DEG_W), lambda i: (1, i, 0)),
            pl.BlockSpec((_R, IN_DIM), lambda i: (i, 0)),
        ],
        out_specs=pl.BlockSpec((2, _R, FW), lambda i: (0, i, 0)),
        out_shape=jax.ShapeDtypeStruct((2, N_NODES, FW), jnp.float32),
    )(deg, deg, x)


def _tc_mid_kernel(p, xs, deg, W1, b1, W2):
    def body(p0, p1, xsr, d0, d1, w1, b1r, w2, o):
        degt = d0[0, :, 0:1] + d1[0, :, 0:1] + 1.0
        dis = lax.rsqrt(degt)
        pr = jnp.concatenate(
            [p0[0] + xsr[0], p1[0] + xsr[1]], axis=1) * dis
        h = jnp.maximum(
            jnp.dot(pr, w1[...], preferred_element_type=jnp.float32) + b1r[...],
            0.0)
        m = jnp.dot(h, w2[...], preferred_element_type=jnp.float32)
        o[0] = m * dis

    return pl.pallas_call(
        body,
        grid=(_G,),
        in_specs=[
            pl.BlockSpec((1, _R, FW), lambda i: (0, i, 0)),
            pl.BlockSpec((1, _R, FW), lambda i: (1, i, 0)),
            pl.BlockSpec((2, _R, FW), lambda i: (0, i, 0)),
            pl.BlockSpec((1, _R, DEG_W), lambda i: (0, i, 0)),
            pl.BlockSpec((1, _R, DEG_W), lambda i: (1, i, 0)),
            pl.BlockSpec((IN_DIM, HID_DIM), lambda i: (0, 0)),
            pl.BlockSpec((1, HID_DIM), lambda i: (0, 0)),
            pl.BlockSpec((HID_DIM, OUT_DIM), lambda i: (0, 0)),
        ],
        out_specs=pl.BlockSpec((1, _R, OUT_DIM), lambda i: (0, i, 0)),
        out_shape=jax.ShapeDtypeStruct((1, N_NODES, OUT_DIM), jnp.float32),
    )(p, p, xs, deg, deg, W1, b1.reshape(1, HID_DIM), W2)


def _tc_final_kernel(q, ms, deg, b2):
    def body(q0, q1, msr, d0, d1, b2r, o):
        degt = d0[0, :, 0:1] + d1[0, :, 0:1] + 1.0
        dis = lax.rsqrt(degt)
        t = (q0[0, 0] + q1[0, 0] + msr[0]) * dis + b2r[...]
        mx = jnp.max(t, axis=1, keepdims=True)
        e = jnp.exp(t - mx)
        lse = jnp.log(jnp.sum(e, axis=1, keepdims=True))
        o[...] = t - mx - lse

    return pl.pallas_call(
        body,
        grid=(_G,),
        in_specs=[
            pl.BlockSpec((1, 1, _R, OUT_DIM), lambda i: (0, 0, i, 0)),
            pl.BlockSpec((1, 1, _R, OUT_DIM), lambda i: (1, 0, i, 0)),
            pl.BlockSpec((1, _R, OUT_DIM), lambda i: (0, i, 0)),
            pl.BlockSpec((1, _R, DEG_W), lambda i: (0, i, 0)),
            pl.BlockSpec((1, _R, DEG_W), lambda i: (1, i, 0)),
            pl.BlockSpec((1, OUT_DIM), lambda i: (0, 0)),
        ],
        out_specs=pl.BlockSpec((_R, OUT_DIM), lambda i: (i, 0)),
        out_shape=jax.ShapeDtypeStruct((N_NODES, OUT_DIM), jnp.float32),
    )(q, q, ms, deg, deg, b2.reshape(1, OUT_DIM))


def kernel(x, edge_index, W1, b1, W2, b2):
    src = edge_index[0].astype(jnp.int32)
    dst = edge_index[1].astype(jnp.int32)

    n_pad = EPW_PAD - EPW
    pad_src = (jnp.arange(n_pad, dtype=jnp.int32) * 89) % N_NODES
    pad_dst = N_NODES + (jnp.arange(n_pad, dtype=jnp.int32) % (NR - N_NODES))
    srcp = jnp.concatenate(
        [src.reshape(NW, EPW), jnp.broadcast_to(pad_src, (NW, n_pad))], axis=1
    ).reshape(NW, CH_IDX, K)
    dstp = jnp.concatenate(
        [dst.reshape(NW, EPW), jnp.broadcast_to(pad_dst, (NW, n_pad))], axis=1
    ).reshape(NW, CH_IDX, K)

    zo = jnp.stack([jnp.zeros((K, DEG_W), jnp.float32),
                    jnp.ones((K, DEG_W), jnp.float32)])
    deg = _deg_kernel()(zo, dst.reshape(TOT_CH, K))
    xs = _tc_scale_kernel(deg, x)
    srcp1 = srcp.reshape(NS, CH1, K)
    dstp1 = dstp.reshape(NS, CH1, K)
    p = _pass1_kernel()(xs, srcp1, dstp1)
    ms = _tc_mid_kernel(p, xs, deg, W1, b1, W2)
    q = _scatter_add_kernel(1)(ms, srcp, dstp)
    return _tc_final_kernel(q, ms, deg, b2)
